# Initial kernel scaffold; baseline (speedup 1.0000x reference)
import jax
import jax.numpy as jnp
from jax.experimental import pallas as pl


def kernel(x, edge_index, W1, b1, W2, b2, W3, b3):
    raise NotImplementedError("write your pallas kernel here")


# speedup vs baseline: 45.0907x; 45.0907x over previous
import functools

import jax
import jax.numpy as jnp
from jax import lax
from jax.experimental import pallas as pl
from jax.experimental.pallas import tpu as pltpu
from jax.experimental.pallas import tpu_sc as plsc

N = 10000
E = 320000
NF = 128
HC1 = 64
HC2 = 32
NCLS = 16

NCORES = 2
NSUB = 16
NW = NCORES * NSUB
LANES = 16

CHUNK = 125
NBUF = 4
N_CHUNKS = E // (NW * CHUNK)
N_PAD = 10240
ROWS_PER_TILE = N_PAD // NSUB
ZROWS = 128

assert NW * N_CHUNKS * CHUNK == E


def _zero_fill(buf, rows, cols):
    zero = jnp.zeros((LANES,), jnp.float32)
    for r in range(rows):
        for c in range(cols // LANES):
            buf[r, pl.ds(c * LANES, LANES)] = zero


@functools.lru_cache(maxsize=None)
def _make_agg(C):
    mesh = plsc.VectorSubcoreMesh(core_axis_name="c", subcore_axis_name="s",
                                  num_cores=NCORES, num_subcores=NSUB)

    @functools.partial(
        pl.kernel,
        mesh=mesh,
        out_type=jax.ShapeDtypeStruct((NCORES, N_PAD, C), jnp.float32),
        compiler_params=pltpu.CompilerParams(use_tc_tiling_on_sc=False),
        scratch_types=[
            pltpu.VMEM((N_CHUNKS, CHUNK), jnp.int32),
            pltpu.VMEM((N_CHUNKS, CHUNK), jnp.int32),
            [pltpu.VMEM((CHUNK, C), jnp.float32) for _ in range(NBUF)],
            pltpu.VMEM((ZROWS, C), jnp.float32),
            pltpu.MemorySpace.VMEM_SHARED((N_PAD, C), jnp.float32),
            pltpu.SemaphoreType.DMA,
            [pltpu.SemaphoreType.DMA for _ in range(NBUF)],
        ],
    )
    def k(hs_hbm, eidx_hbm, out_hbm, sidx, didx, rows, zbuf, acc, isem, bsem):
        cid = lax.axis_index("c")
        sid = lax.axis_index("s")
        wid = cid * NSUB + sid
        r0 = wid * N_CHUNKS
        sload = pltpu.async_copy(eidx_hbm.at[0, pl.ds(r0, N_CHUNKS)], sidx,
                                 isem)
        dload = pltpu.async_copy(eidx_hbm.at[1, pl.ds(r0, N_CHUNKS)], didx,
                                 isem)
        _zero_fill(zbuf, ZROWS, C)
        row0 = sid * ROWS_PER_TILE

        def zloop(i, carry):
            pltpu.sync_copy(zbuf, acc.at[pl.ds(row0 + i * ZROWS, ZROWS)])
            return carry

        lax.fori_loop(0, ROWS_PER_TILE // ZROWS, zloop, 0)
        sload.wait()
        dload.wait()
        plsc.subcore_barrier()

        def gather(i, b):
            pltpu.async_copy(hs_hbm.at[sidx.at[i]], rows[b], bsem[b])

        def scatter(i, b):
            pltpu.async_copy(rows[b], acc.at[didx.at[i]], bsem[b], add=True)

        def wait_b(b):
            pltpu.make_async_copy(hs_hbm.at[sidx.at[0]], rows[b],
                                  bsem[b]).wait()

        def body(g, carry):
            i0 = g * NBUF
            for b in range(NBUF):
                pl.when(g > 0)(lambda b=b: wait_b(b))
                gather(i0 + b, b)
            for b in range(NBUF):
                wait_b(b)
                scatter(i0 + b, b)
            return carry

        lax.fori_loop(0, N_CHUNKS // NBUF, body, 0)
        for b in range(NBUF):
            wait_b(b)
        plsc.subcore_barrier()
        pltpu.sync_copy(
            acc.at[pl.ds(row0, ROWS_PER_TILE)],
            out_hbm.at[cid, pl.ds(row0, ROWS_PER_TILE)],
        )

    return k


@functools.lru_cache(maxsize=None)
def _make_deg():
    C = 16
    mesh = plsc.VectorSubcoreMesh(core_axis_name="c", subcore_axis_name="s",
                                  num_cores=NCORES, num_subcores=NSUB)

    @functools.partial(
        pl.kernel,
        mesh=mesh,
        out_type=jax.ShapeDtypeStruct((NCORES, N_PAD, C), jnp.float32),
        compiler_params=pltpu.CompilerParams(use_tc_tiling_on_sc=False),
        scratch_types=[
            pltpu.VMEM((N_CHUNKS, CHUNK), jnp.int32),
            pltpu.VMEM((CHUNK, C), jnp.float32),
            pltpu.VMEM((ZROWS, C), jnp.float32),
            pltpu.MemorySpace.VMEM_SHARED((N_PAD, C), jnp.float32),
            pltpu.SemaphoreType.DMA,
            [pltpu.SemaphoreType.DMA for _ in range(NBUF)],
        ],
    )
    def k(eidx_hbm, out_hbm, didx, ones, zbuf, acc, isem, bsem):
        cid = lax.axis_index("c")
        sid = lax.axis_index("s")
        wid = cid * NSUB + sid
        r0 = wid * N_CHUNKS
        dload = pltpu.async_copy(eidx_hbm.at[1, pl.ds(r0, N_CHUNKS)], didx,
                                 isem)
        _zero_fill(zbuf, ZROWS, C)
        one = jnp.ones((LANES,), jnp.float32)
        for r in range(CHUNK):
            ones[r, pl.ds(0, LANES)] = one
        row0 = sid * ROWS_PER_TILE

        def zloop(i, carry):
            pltpu.sync_copy(zbuf, acc.at[pl.ds(row0 + i * ZROWS, ZROWS)])
            return carry

        lax.fori_loop(0, ROWS_PER_TILE // ZROWS, zloop, 0)
        dload.wait()
        plsc.subcore_barrier()

        def wait_b(b):
            pltpu.make_async_copy(ones, acc.at[didx.at[0]], bsem[b]).wait()

        def body(g, carry):
            i0 = g * NBUF
            for b in range(NBUF):
                pl.when(g > 0)(lambda b=b: wait_b(b))
                pltpu.async_copy(ones, acc.at[didx.at[i0 + b]], bsem[b],
                                 add=True)
            return carry

        lax.fori_loop(0, N_CHUNKS // NBUF, body, 0)
        for b in range(NBUF):
            wait_b(b)
        plsc.subcore_barrier()
        pltpu.sync_copy(
            acc.at[pl.ds(row0, ROWS_PER_TILE)],
            out_hbm.at[cid, pl.ds(row0, ROWS_PER_TILE)],
        )

    return k


BN = 1024
GRID = N_PAD // BN


def _r128(n_rows, C):
    return (n_rows * C // 128, 128)


def _tc_mm0(x, W):
    def body(x_ref, w_ref, o_ref):
        o_ref[...] = jnp.dot(x_ref[...], w_ref[...],
                             preferred_element_type=jnp.float32)

    return pl.pallas_call(
        body,
        grid=(GRID,),
        in_specs=[
            pl.BlockSpec((BN, NF), lambda i: (i, 0)),
            pl.BlockSpec((NF, HC1), lambda i: (0, 0)),
        ],
        out_specs=pl.BlockSpec((BN, HC1), lambda i: (i, 0)),
        out_shape=jax.ShapeDtypeStruct((N_PAD, HC1), jnp.float32),
    )(x, W)


def _tc_scale0(h1raw, degp):
    def body(h_ref, d_ref, dis_ref, hs_ref):
        d = d_ref[...]
        dis = lax.rsqrt(1.0 + d[0, :, 0:1] + d[1, :, 0:1])
        dis_ref[...] = dis
        hs_ref[...] = h_ref[...] * dis

    return pl.pallas_call(
        body,
        grid=(GRID,),
        in_specs=[
            pl.BlockSpec((BN, HC1), lambda i: (i, 0)),
            pl.BlockSpec((NCORES, BN, 16), lambda i: (0, i, 0)),
        ],
        out_specs=[
            pl.BlockSpec((BN, 1), lambda i: (i, 0)),
            pl.BlockSpec((BN, HC1), lambda i: (i, 0)),
        ],
        out_shape=[
            jax.ShapeDtypeStruct((N_PAD, 1), jnp.float32),
            jax.ShapeDtypeStruct((N_PAD, HC1), jnp.float32),
        ],
    )(h1raw, degp)


def _tc_mid(aggp, hs, dis, b, W, C, C2):
    def body(a_ref, hs_ref, dis_ref, b_ref, w_ref, o_ref):
        a = a_ref[...]
        s = a[0] + a[1] + hs_ref[...]
        dis = dis_ref[...]
        t = jnp.tanh(s * dis + b_ref[...])
        o_ref[...] = jnp.dot(t, w_ref[...],
                             preferred_element_type=jnp.float32) * dis

    return pl.pallas_call(
        body,
        grid=(GRID,),
        in_specs=[
            pl.BlockSpec((NCORES, BN, C), lambda i: (0, i, 0)),
            pl.BlockSpec((BN, C), lambda i: (i, 0)),
            pl.BlockSpec((BN, 1), lambda i: (i, 0)),
            pl.BlockSpec((1, C), lambda i: (0, 0)),
            pl.BlockSpec((C, C2), lambda i: (0, 0)),
        ],
        out_specs=pl.BlockSpec((BN, C2), lambda i: (i, 0)),
        out_shape=jax.ShapeDtypeStruct((N_PAD, C2), jnp.float32),
    )(aggp, hs, dis, b, W)


def _tc_emb(aggp, hs, dis, b, W):
    def body(a_ref, hs_ref, dis_ref, b_ref, w_ref, emb_ref, o_ref):
        a = a_ref[...]
        s = a[0] + a[1] + hs_ref[...]
        dis = dis_ref[...]
        emb = s * dis + b_ref[...]
        emb_ref[...] = emb
        o_ref[...] = jnp.dot(jnp.tanh(emb), w_ref[...],
                             preferred_element_type=jnp.float32) * dis

    return pl.pallas_call(
        body,
        grid=(GRID,),
        in_specs=[
            pl.BlockSpec((NCORES, BN, HC2), lambda i: (0, i, 0)),
            pl.BlockSpec((BN, HC2), lambda i: (i, 0)),
            pl.BlockSpec((BN, 1), lambda i: (i, 0)),
            pl.BlockSpec((1, HC2), lambda i: (0, 0)),
            pl.BlockSpec((HC2, NCLS), lambda i: (0, 0)),
        ],
        out_specs=[
            pl.BlockSpec((BN, HC2), lambda i: (i, 0)),
            pl.BlockSpec((BN, NCLS), lambda i: (i, 0)),
        ],
        out_shape=[
            jax.ShapeDtypeStruct((N_PAD, HC2), jnp.float32),
            jax.ShapeDtypeStruct((N_PAD, NCLS), jnp.float32),
        ],
    )(aggp, hs, dis, b, W)


def _tc_final(aggp, hs, dis, b):
    def body(a_ref, hs_ref, dis_ref, b_ref, o_ref):
        a = a_ref[...]
        s = a[0] + a[1] + hs_ref[...]
        o_ref[...] = s * dis_ref[...] + b_ref[...]

    return pl.pallas_call(
        body,
        grid=(GRID,),
        in_specs=[
            pl.BlockSpec((NCORES, BN, NCLS), lambda i: (0, i, 0)),
            pl.BlockSpec((BN, NCLS), lambda i: (i, 0)),
            pl.BlockSpec((BN, 1), lambda i: (i, 0)),
            pl.BlockSpec((1, NCLS), lambda i: (0, 0)),
        ],
        out_specs=pl.BlockSpec((BN, NCLS), lambda i: (i, 0)),
        out_shape=jax.ShapeDtypeStruct((N_PAD, NCLS), jnp.float32),
    )(aggp, hs, dis, b)


def _v128(a, C):
    return a.reshape(NCORES, N_PAD * C // 128, 128)


def kernel(x, edge_index, W1, b1, W2, b2, W3, b3):
    eidx = edge_index.reshape(2, NW * N_CHUNKS, CHUNK)
    x_p = jnp.pad(x, ((0, N_PAD - N), (0, 0)))

    degp = _make_deg()(eidx)
    h1raw = _tc_mm0(x_p, W1)
    dis, hs1 = _tc_scale0(h1raw, degp)

    aggp1 = _make_agg(HC1)(hs1, eidx)
    hs2 = _tc_mid(aggp1, hs1, dis, b1.reshape(1, HC1), W2, HC1, HC2)

    aggp2 = _make_agg(HC2)(hs2, eidx)
    emb_p, hs3 = _tc_emb(aggp2, hs2, dis, b2.reshape(1, HC2), W3)

    aggp3 = _make_agg(NCLS)(hs3, eidx)
    logits_p = _tc_final(aggp3, hs3, dis, b3.reshape(1, NCLS))

    return (logits_p[:N], emb_p[:N])


# speedup vs baseline: 47.8825x; 1.0619x over previous
import functools

import jax
import jax.numpy as jnp
from jax import lax
from jax.experimental import pallas as pl
from jax.experimental.pallas import tpu as pltpu
from jax.experimental.pallas import tpu_sc as plsc

N = 10000
E = 320000
NF = 128
HC1 = 64
HC2 = 32
NCLS = 16

NCORES = 2
NSUB = 16
NW = NCORES * NSUB
LANES = 16

CHUNK = 125
NBUF = 5
N_CHUNKS = E // (NW * CHUNK)
N_PAD = 10240
ROWS_PER_TILE = N_PAD // NSUB
ZROWS = 128

assert NW * N_CHUNKS * CHUNK == E


def _zero_fill(buf, rows, cols):
    zero = jnp.zeros((LANES,), jnp.float32)
    for r in range(rows):
        for c in range(cols // LANES):
            buf[r, pl.ds(c * LANES, LANES)] = zero


@functools.lru_cache(maxsize=None)
def _make_agg(C):
    mesh = plsc.VectorSubcoreMesh(core_axis_name="c", subcore_axis_name="s",
                                  num_cores=NCORES, num_subcores=NSUB)

    @functools.partial(
        pl.kernel,
        mesh=mesh,
        out_type=jax.ShapeDtypeStruct((NCORES, N_PAD, C), jnp.float32),
        compiler_params=pltpu.CompilerParams(use_tc_tiling_on_sc=False),
        scratch_types=[
            pltpu.VMEM((N_CHUNKS, CHUNK), jnp.int32),
            pltpu.VMEM((N_CHUNKS, CHUNK), jnp.int32),
            [pltpu.VMEM((CHUNK, C), jnp.float32) for _ in range(NBUF)],
            pltpu.VMEM((ZROWS, C), jnp.float32),
            pltpu.MemorySpace.VMEM_SHARED((N_PAD, C), jnp.float32),
            pltpu.SemaphoreType.DMA,
            [pltpu.SemaphoreType.DMA for _ in range(NBUF)],
        ],
    )
    def k(hs_hbm, eidx_hbm, out_hbm, sidx, didx, rows, zbuf, acc, isem, bsem):
        cid = lax.axis_index("c")
        sid = lax.axis_index("s")
        wid = cid * NSUB + sid
        r0 = wid * N_CHUNKS
        sload = pltpu.async_copy(eidx_hbm.at[0, pl.ds(r0, N_CHUNKS)], sidx,
                                 isem)
        dload = pltpu.async_copy(eidx_hbm.at[1, pl.ds(r0, N_CHUNKS)], didx,
                                 isem)
        _zero_fill(zbuf, ZROWS, C)
        row0 = sid * ROWS_PER_TILE

        def zloop(i, carry):
            pltpu.sync_copy(zbuf, acc.at[pl.ds(row0 + i * ZROWS, ZROWS)])
            return carry

        lax.fori_loop(0, ROWS_PER_TILE // ZROWS, zloop, 0)
        sload.wait()
        dload.wait()
        plsc.subcore_barrier()

        def gather(i, b):
            pltpu.async_copy(hs_hbm.at[sidx.at[i]], rows[b], bsem[b])

        def scatter(i, b):
            pltpu.async_copy(rows[b], acc.at[didx.at[i]], bsem[b], add=True)

        def wait_b(b):
            pltpu.make_async_copy(hs_hbm.at[sidx.at[0]], rows[b],
                                  bsem[b]).wait()

        def body(g, carry):
            i0 = g * NBUF
            for b in range(NBUF):
                pl.when(g > 0)(lambda b=b: wait_b(b))
                gather(i0 + b, b)
            for b in range(NBUF):
                wait_b(b)
                scatter(i0 + b, b)
            return carry

        lax.fori_loop(0, N_CHUNKS // NBUF, body, 0)
        for b in range(NBUF):
            wait_b(b)
        plsc.subcore_barrier()
        pltpu.sync_copy(
            acc.at[pl.ds(row0, ROWS_PER_TILE)],
            out_hbm.at[cid, pl.ds(row0, ROWS_PER_TILE)],
        )

    return k


@functools.lru_cache(maxsize=None)
def _make_deg():
    C = 16
    mesh = plsc.VectorSubcoreMesh(core_axis_name="c", subcore_axis_name="s",
                                  num_cores=NCORES, num_subcores=NSUB)

    @functools.partial(
        pl.kernel,
        mesh=mesh,
        out_type=jax.ShapeDtypeStruct((NCORES, N_PAD, C), jnp.float32),
        compiler_params=pltpu.CompilerParams(use_tc_tiling_on_sc=False),
        scratch_types=[
            pltpu.VMEM((N_CHUNKS, CHUNK), jnp.int32),
            pltpu.VMEM((CHUNK, C), jnp.float32),
            pltpu.VMEM((ZROWS, C), jnp.float32),
            pltpu.MemorySpace.VMEM_SHARED((N_PAD, C), jnp.float32),
            pltpu.SemaphoreType.DMA,
            [pltpu.SemaphoreType.DMA for _ in range(NBUF)],
        ],
    )
    def k(eidx_hbm, out_hbm, didx, ones, zbuf, acc, isem, bsem):
        cid = lax.axis_index("c")
        sid = lax.axis_index("s")
        wid = cid * NSUB + sid
        r0 = wid * N_CHUNKS
        dload = pltpu.async_copy(eidx_hbm.at[1, pl.ds(r0, N_CHUNKS)], didx,
                                 isem)
        _zero_fill(zbuf, ZROWS, C)
        one = jnp.ones((LANES,), jnp.float32)
        for r in range(CHUNK):
            ones[r, pl.ds(0, LANES)] = one
        row0 = sid * ROWS_PER_TILE

        def zloop(i, carry):
            pltpu.sync_copy(zbuf, acc.at[pl.ds(row0 + i * ZROWS, ZROWS)])
            return carry

        lax.fori_loop(0, ROWS_PER_TILE // ZROWS, zloop, 0)
        dload.wait()
        plsc.subcore_barrier()

        def wait_b(b):
            pltpu.make_async_copy(ones, acc.at[didx.at[0]], bsem[b]).wait()

        def body(g, carry):
            i0 = g * NBUF
            for b in range(NBUF):
                pl.when(g > 0)(lambda b=b: wait_b(b))
                pltpu.async_copy(ones, acc.at[didx.at[i0 + b]], bsem[b],
                                 add=True)
            return carry

        lax.fori_loop(0, N_CHUNKS // NBUF, body, 0)
        for b in range(NBUF):
            wait_b(b)
        plsc.subcore_barrier()
        pltpu.sync_copy(
            acc.at[pl.ds(row0, ROWS_PER_TILE)],
            out_hbm.at[cid, pl.ds(row0, ROWS_PER_TILE)],
        )

    return k


BN = 2048
GRID = N_PAD // BN


def _r128(n_rows, C):
    return (n_rows * C // 128, 128)


def _tc_mm0(x, W):
    def body(x_ref, w_ref, o_ref):
        o_ref[...] = jnp.dot(x_ref[...], w_ref[...],
                             preferred_element_type=jnp.float32)

    return pl.pallas_call(
        body,
        grid=(GRID,),
        in_specs=[
            pl.BlockSpec((BN, NF), lambda i: (i, 0)),
            pl.BlockSpec((NF, HC1), lambda i: (0, 0)),
        ],
        out_specs=pl.BlockSpec((BN, HC1), lambda i: (i, 0)),
        out_shape=jax.ShapeDtypeStruct((N_PAD, HC1), jnp.float32),
    )(x, W)


def _tc_scale0(h1raw, degp):
    def body(h_ref, d_ref, dis_ref, hs_ref):
        d = d_ref[...]
        dis = lax.rsqrt(1.0 + d[0, :, 0:1] + d[1, :, 0:1])
        dis_ref[...] = dis
        hs_ref[...] = h_ref[...] * dis

    return pl.pallas_call(
        body,
        grid=(GRID,),
        in_specs=[
            pl.BlockSpec((BN, HC1), lambda i: (i, 0)),
            pl.BlockSpec((NCORES, BN, 16), lambda i: (0, i, 0)),
        ],
        out_specs=[
            pl.BlockSpec((BN, 1), lambda i: (i, 0)),
            pl.BlockSpec((BN, HC1), lambda i: (i, 0)),
        ],
        out_shape=[
            jax.ShapeDtypeStruct((N_PAD, 1), jnp.float32),
            jax.ShapeDtypeStruct((N_PAD, HC1), jnp.float32),
        ],
    )(h1raw, degp)


def _tc_mid(aggp, hs, dis, b, W, C, C2):
    def body(a_ref, hs_ref, dis_ref, b_ref, w_ref, o_ref):
        a = a_ref[...]
        s = a[0] + a[1] + hs_ref[...]
        dis = dis_ref[...]
        t = jnp.tanh(s * dis + b_ref[...])
        o_ref[...] = jnp.dot(t, w_ref[...],
                             preferred_element_type=jnp.float32) * dis

    return pl.pallas_call(
        body,
        grid=(GRID,),
        in_specs=[
            pl.BlockSpec((NCORES, BN, C), lambda i: (0, i, 0)),
            pl.BlockSpec((BN, C), lambda i: (i, 0)),
            pl.BlockSpec((BN, 1), lambda i: (i, 0)),
            pl.BlockSpec((1, C), lambda i: (0, 0)),
            pl.BlockSpec((C, C2), lambda i: (0, 0)),
        ],
        out_specs=pl.BlockSpec((BN, C2), lambda i: (i, 0)),
        out_shape=jax.ShapeDtypeStruct((N_PAD, C2), jnp.float32),
    )(aggp, hs, dis, b, W)


def _tc_emb(aggp, hs, dis, b, W):
    def body(a_ref, hs_ref, dis_ref, b_ref, w_ref, emb_ref, o_ref):
        a = a_ref[...]
        s = a[0] + a[1] + hs_ref[...]
        dis = dis_ref[...]
        emb = s * dis + b_ref[...]
        emb_ref[...] = emb
        o_ref[...] = jnp.dot(jnp.tanh(emb), w_ref[...],
                             preferred_element_type=jnp.float32) * dis

    return pl.pallas_call(
        body,
        grid=(GRID,),
        in_specs=[
            pl.BlockSpec((NCORES, BN, HC2), lambda i: (0, i, 0)),
            pl.BlockSpec((BN, HC2), lambda i: (i, 0)),
            pl.BlockSpec((BN, 1), lambda i: (i, 0)),
            pl.BlockSpec((1, HC2), lambda i: (0, 0)),
            pl.BlockSpec((HC2, NCLS), lambda i: (0, 0)),
        ],
        out_specs=[
            pl.BlockSpec((BN, HC2), lambda i: (i, 0)),
            pl.BlockSpec((BN, NCLS), lambda i: (i, 0)),
        ],
        out_shape=[
            jax.ShapeDtypeStruct((N_PAD, HC2), jnp.float32),
            jax.ShapeDtypeStruct((N_PAD, NCLS), jnp.float32),
        ],
    )(aggp, hs, dis, b, W)


def _tc_final(aggp, hs, dis, b):
    def body(a_ref, hs_ref, dis_ref, b_ref, o_ref):
        a = a_ref[...]
        s = a[0] + a[1] + hs_ref[...]
        o_ref[...] = s * dis_ref[...] + b_ref[...]

    return pl.pallas_call(
        body,
        grid=(GRID,),
        in_specs=[
            pl.BlockSpec((NCORES, BN, NCLS), lambda i: (0, i, 0)),
            pl.BlockSpec((BN, NCLS), lambda i: (i, 0)),
            pl.BlockSpec((BN, 1), lambda i: (i, 0)),
            pl.BlockSpec((1, NCLS), lambda i: (0, 0)),
        ],
        out_specs=pl.BlockSpec((BN, NCLS), lambda i: (i, 0)),
        out_shape=jax.ShapeDtypeStruct((N_PAD, NCLS), jnp.float32),
    )(aggp, hs, dis, b)


def _v128(a, C):
    return a.reshape(NCORES, N_PAD * C // 128, 128)


def kernel(x, edge_index, W1, b1, W2, b2, W3, b3):
    eidx = edge_index.reshape(2, NW * N_CHUNKS, CHUNK)
    x_p = jnp.pad(x, ((0, N_PAD - N), (0, 0)))

    degp = _make_deg()(eidx)
    h1raw = _tc_mm0(x_p, W1)
    dis, hs1 = _tc_scale0(h1raw, degp)

    aggp1 = _make_agg(HC1)(hs1, eidx)
    hs2 = _tc_mid(aggp1, hs1, dis, b1.reshape(1, HC1), W2, HC1, HC2)

    aggp2 = _make_agg(HC2)(hs2, eidx)
    emb_p, hs3 = _tc_emb(aggp2, hs2, dis, b2.reshape(1, HC2), W3)

    aggp3 = _make_agg(NCLS)(hs3, eidx)
    logits_p = _tc_final(aggp3, hs3, dis, b3.reshape(1, NCLS))

    return (logits_p[:N], emb_p[:N])


# speedup vs baseline: 48.7829x; 1.0188x over previous
import functools

import jax
import jax.numpy as jnp
from jax import lax
from jax.experimental import pallas as pl
from jax.experimental.pallas import tpu as pltpu
from jax.experimental.pallas import tpu_sc as plsc

N = 10000
E = 320000
NF = 128
HC1 = 64
HC2 = 32
NCLS = 16

NCORES = 2
NSUB = 16
NW = NCORES * NSUB
LANES = 16

CHUNK = 125
NBUF = 5
N_CHUNKS = E // (NW * CHUNK)
N_PAD = 10240
ROWS_PER_TILE = N_PAD // NSUB
ZROWS = 128

assert NW * N_CHUNKS * CHUNK == E


def _zero_fill(buf, rows, cols):
    zero = jnp.zeros((LANES,), jnp.float32)
    for r in range(rows):
        for c in range(cols // LANES):
            buf[r, pl.ds(c * LANES, LANES)] = zero


@functools.lru_cache(maxsize=None)
def _make_agg(C):
    NB = NBUF if C >= 64 else 10
    mesh = plsc.VectorSubcoreMesh(core_axis_name="c", subcore_axis_name="s",
                                  num_cores=NCORES, num_subcores=NSUB)

    @functools.partial(
        pl.kernel,
        mesh=mesh,
        out_type=jax.ShapeDtypeStruct((NCORES, N_PAD, C), jnp.float32),
        compiler_params=pltpu.CompilerParams(use_tc_tiling_on_sc=False),
        scratch_types=[
            pltpu.VMEM((N_CHUNKS, CHUNK), jnp.int32),
            pltpu.VMEM((N_CHUNKS, CHUNK), jnp.int32),
            [pltpu.VMEM((CHUNK, C), jnp.float32) for _ in range(NB)],
            pltpu.VMEM((ZROWS, C), jnp.float32),
            pltpu.MemorySpace.VMEM_SHARED((N_PAD, C), jnp.float32),
            pltpu.SemaphoreType.DMA,
            [pltpu.SemaphoreType.DMA for _ in range(NB)],
        ],
    )
    def k(hs_hbm, eidx_hbm, out_hbm, sidx, didx, rows, zbuf, acc, isem, bsem):
        cid = lax.axis_index("c")
        sid = lax.axis_index("s")
        wid = cid * NSUB + sid
        r0 = wid * N_CHUNKS
        sload = pltpu.async_copy(eidx_hbm.at[0, pl.ds(r0, N_CHUNKS)], sidx,
                                 isem)
        dload = pltpu.async_copy(eidx_hbm.at[1, pl.ds(r0, N_CHUNKS)], didx,
                                 isem)
        _zero_fill(zbuf, ZROWS, C)
        row0 = sid * ROWS_PER_TILE

        def zloop(i, carry):
            pltpu.sync_copy(zbuf, acc.at[pl.ds(row0 + i * ZROWS, ZROWS)])
            return carry

        lax.fori_loop(0, ROWS_PER_TILE // ZROWS, zloop, 0)
        sload.wait()
        dload.wait()
        plsc.subcore_barrier()

        def gather(i, b):
            pltpu.async_copy(hs_hbm.at[sidx.at[i]], rows[b], bsem[b])

        def scatter(i, b):
            pltpu.async_copy(rows[b], acc.at[didx.at[i]], bsem[b], add=True)

        def wait_b(b):
            pltpu.make_async_copy(hs_hbm.at[sidx.at[0]], rows[b],
                                  bsem[b]).wait()

        def body(g, carry):
            i0 = g * NB
            for b in range(NB):
                pl.when(g > 0)(lambda b=b: wait_b(b))
                gather(i0 + b, b)
            for b in range(NB):
                wait_b(b)
                scatter(i0 + b, b)
            return carry

        lax.fori_loop(0, N_CHUNKS // NB, body, 0)
        for b in range(NB):
            wait_b(b)
        plsc.subcore_barrier()
        pltpu.sync_copy(
            acc.at[pl.ds(row0, ROWS_PER_TILE)],
            out_hbm.at[cid, pl.ds(row0, ROWS_PER_TILE)],
        )

    return k


@functools.lru_cache(maxsize=None)
def _make_deg():
    C = 16
    mesh = plsc.VectorSubcoreMesh(core_axis_name="c", subcore_axis_name="s",
                                  num_cores=NCORES, num_subcores=NSUB)

    @functools.partial(
        pl.kernel,
        mesh=mesh,
        out_type=jax.ShapeDtypeStruct((NCORES, N_PAD, C), jnp.float32),
        compiler_params=pltpu.CompilerParams(use_tc_tiling_on_sc=False),
        scratch_types=[
            pltpu.VMEM((N_CHUNKS, CHUNK), jnp.int32),
            pltpu.VMEM((CHUNK, C), jnp.float32),
            pltpu.VMEM((ZROWS, C), jnp.float32),
            pltpu.MemorySpace.VMEM_SHARED((N_PAD, C), jnp.float32),
            pltpu.SemaphoreType.DMA,
            [pltpu.SemaphoreType.DMA for _ in range(NBUF)],
        ],
    )
    def k(eidx_hbm, out_hbm, didx, ones, zbuf, acc, isem, bsem):
        cid = lax.axis_index("c")
        sid = lax.axis_index("s")
        wid = cid * NSUB + sid
        r0 = wid * N_CHUNKS
        dload = pltpu.async_copy(eidx_hbm.at[1, pl.ds(r0, N_CHUNKS)], didx,
                                 isem)
        _zero_fill(zbuf, ZROWS, C)
        one = jnp.ones((LANES,), jnp.float32)
        for r in range(CHUNK):
            ones[r, pl.ds(0, LANES)] = one
        row0 = sid * ROWS_PER_TILE

        def zloop(i, carry):
            pltpu.sync_copy(zbuf, acc.at[pl.ds(row0 + i * ZROWS, ZROWS)])
            return carry

        lax.fori_loop(0, ROWS_PER_TILE // ZROWS, zloop, 0)
        dload.wait()
        plsc.subcore_barrier()

        def wait_b(b):
            pltpu.make_async_copy(ones, acc.at[didx.at[0]], bsem[b]).wait()

        def body(g, carry):
            i0 = g * NBUF
            for b in range(NBUF):
                pl.when(g > 0)(lambda b=b: wait_b(b))
                pltpu.async_copy(ones, acc.at[didx.at[i0 + b]], bsem[b],
                                 add=True)
            return carry

        lax.fori_loop(0, N_CHUNKS // NBUF, body, 0)
        for b in range(NBUF):
            wait_b(b)
        plsc.subcore_barrier()
        pltpu.sync_copy(
            acc.at[pl.ds(row0, ROWS_PER_TILE)],
            out_hbm.at[cid, pl.ds(row0, ROWS_PER_TILE)],
        )

    return k


BN = 2048
GRID = N_PAD // BN


def _r128(n_rows, C):
    return (n_rows * C // 128, 128)


def _tc_mm0(x, W):
    def body(x_ref, w_ref, o_ref):
        o_ref[...] = jnp.dot(x_ref[...], w_ref[...],
                             preferred_element_type=jnp.float32)

    return pl.pallas_call(
        body,
        grid=(GRID,),
        in_specs=[
            pl.BlockSpec((BN, NF), lambda i: (i, 0)),
            pl.BlockSpec((NF, HC1), lambda i: (0, 0)),
        ],
        out_specs=pl.BlockSpec((BN, HC1), lambda i: (i, 0)),
        out_shape=jax.ShapeDtypeStruct((N_PAD, HC1), jnp.float32),
    )(x, W)


def _tc_scale0(h1raw, degp):
    def body(h_ref, d_ref, dis_ref, hs_ref):
        d = d_ref[...]
        dis = lax.rsqrt(1.0 + d[0, :, 0:1] + d[1, :, 0:1])
        dis_ref[...] = dis
        hs_ref[...] = h_ref[...] * dis

    return pl.pallas_call(
        body,
        grid=(GRID,),
        in_specs=[
            pl.BlockSpec((BN, HC1), lambda i: (i, 0)),
            pl.BlockSpec((NCORES, BN, 16), lambda i: (0, i, 0)),
        ],
        out_specs=[
            pl.BlockSpec((BN, 1), lambda i: (i, 0)),
            pl.BlockSpec((BN, HC1), lambda i: (i, 0)),
        ],
        out_shape=[
            jax.ShapeDtypeStruct((N_PAD, 1), jnp.float32),
            jax.ShapeDtypeStruct((N_PAD, HC1), jnp.float32),
        ],
    )(h1raw, degp)


def _tc_mid(aggp, hs, dis, b, W, C, C2):
    def body(a_ref, hs_ref, dis_ref, b_ref, w_ref, o_ref):
        a = a_ref[...]
        s = a[0] + a[1] + hs_ref[...]
        dis = dis_ref[...]
        t = jnp.tanh(s * dis + b_ref[...])
        o_ref[...] = jnp.dot(t, w_ref[...],
                             preferred_element_type=jnp.float32) * dis

    return pl.pallas_call(
        body,
        grid=(GRID,),
        in_specs=[
            pl.BlockSpec((NCORES, BN, C), lambda i: (0, i, 0)),
            pl.BlockSpec((BN, C), lambda i: (i, 0)),
            pl.BlockSpec((BN, 1), lambda i: (i, 0)),
            pl.BlockSpec((1, C), lambda i: (0, 0)),
            pl.BlockSpec((C, C2), lambda i: (0, 0)),
        ],
        out_specs=pl.BlockSpec((BN, C2), lambda i: (i, 0)),
        out_shape=jax.ShapeDtypeStruct((N_PAD, C2), jnp.float32),
    )(aggp, hs, dis, b, W)


def _tc_emb(aggp, hs, dis, b, W):
    def body(a_ref, hs_ref, dis_ref, b_ref, w_ref, emb_ref, o_ref):
        a = a_ref[...]
        s = a[0] + a[1] + hs_ref[...]
        dis = dis_ref[...]
        emb = s * dis + b_ref[...]
        emb_ref[...] = emb
        o_ref[...] = jnp.dot(jnp.tanh(emb), w_ref[...],
                             preferred_element_type=jnp.float32) * dis

    return pl.pallas_call(
        body,
        grid=(GRID,),
        in_specs=[
            pl.BlockSpec((NCORES, BN, HC2), lambda i: (0, i, 0)),
            pl.BlockSpec((BN, HC2), lambda i: (i, 0)),
            pl.BlockSpec((BN, 1), lambda i: (i, 0)),
            pl.BlockSpec((1, HC2), lambda i: (0, 0)),
            pl.BlockSpec((HC2, NCLS), lambda i: (0, 0)),
        ],
        out_specs=[
            pl.BlockSpec((BN, HC2), lambda i: (i, 0)),
            pl.BlockSpec((BN, NCLS), lambda i: (i, 0)),
        ],
        out_shape=[
            jax.ShapeDtypeStruct((N_PAD, HC2), jnp.float32),
            jax.ShapeDtypeStruct((N_PAD, NCLS), jnp.float32),
        ],
    )(aggp, hs, dis, b, W)


def _tc_final(aggp, hs, dis, b):
    def body(a_ref, hs_ref, dis_ref, b_ref, o_ref):
        a = a_ref[...]
        s = a[0] + a[1] + hs_ref[...]
        o_ref[...] = s * dis_ref[...] + b_ref[...]

    return pl.pallas_call(
        body,
        grid=(GRID,),
        in_specs=[
            pl.BlockSpec((NCORES, BN, NCLS), lambda i: (0, i, 0)),
            pl.BlockSpec((BN, NCLS), lambda i: (i, 0)),
            pl.BlockSpec((BN, 1), lambda i: (i, 0)),
            pl.BlockSpec((1, NCLS), lambda i: (0, 0)),
        ],
        out_specs=pl.BlockSpec((BN, NCLS), lambda i: (i, 0)),
        out_shape=jax.ShapeDtypeStruct((N_PAD, NCLS), jnp.float32),
    )(aggp, hs, dis, b)


def _v128(a, C):
    return a.reshape(NCORES, N_PAD * C // 128, 128)


def kernel(x, edge_index, W1, b1, W2, b2, W3, b3):
    eidx = edge_index.reshape(2, NW * N_CHUNKS, CHUNK)
    x_p = jnp.pad(x, ((0, N_PAD - N), (0, 0)))

    degp = _make_deg()(eidx)
    h1raw = _tc_mm0(x_p, W1)
    dis, hs1 = _tc_scale0(h1raw, degp)

    aggp1 = _make_agg(HC1)(hs1, eidx)
    hs2 = _tc_mid(aggp1, hs1, dis, b1.reshape(1, HC1), W2, HC1, HC2)

    aggp2 = _make_agg(HC2)(hs2, eidx)
    emb_p, hs3 = _tc_emb(aggp2, hs2, dis, b2.reshape(1, HC2), W3)

    aggp3 = _make_agg(NCLS)(hs3, eidx)
    logits_p = _tc_final(aggp3, hs3, dis, b3.reshape(1, NCLS))

    return (logits_p[:N], emb_p[:N])


# speedup vs baseline: 54.3336x; 1.1138x over previous
import functools

import jax
import jax.numpy as jnp
from jax import lax
from jax.experimental import pallas as pl
from jax.experimental.pallas import tpu as pltpu
from jax.experimental.pallas import tpu_sc as plsc

N = 10000
E = 320000
NF = 128
HC1 = 64
HC2 = 32
NCLS = 16

NCORES = 2
NSUB = 16
NW = NCORES * NSUB
LANES = 16

CHUNK = 125
NBUF = 5
N_CHUNKS = E // (NW * CHUNK)
N_PAD = 10240
ROWS_PER_TILE = N_PAD // NSUB
ZROWS = 128

assert NW * N_CHUNKS * CHUNK == E


def _zero_fill(buf, rows, cols):
    zero = jnp.zeros((LANES,), jnp.float32)
    for r in range(rows):
        for c in range(cols // LANES):
            buf[r, pl.ds(c * LANES, LANES)] = zero


@functools.lru_cache(maxsize=None)
def _make_agg(C):
    NB = NBUF if C >= 64 else 10
    mesh = plsc.VectorSubcoreMesh(core_axis_name="c", subcore_axis_name="s",
                                  num_cores=NCORES, num_subcores=NSUB)

    @functools.partial(
        pl.kernel,
        mesh=mesh,
        out_type=jax.ShapeDtypeStruct((N_PAD, 128), jnp.float32),
        compiler_params=pltpu.CompilerParams(use_tc_tiling_on_sc=False),
        scratch_types=[
            pltpu.VMEM((N_CHUNKS, CHUNK), jnp.int32),
            pltpu.VMEM((N_CHUNKS, CHUNK), jnp.int32),
            [pltpu.VMEM((CHUNK, C), jnp.float32) for _ in range(NB)],
            pltpu.VMEM((ZROWS, C), jnp.float32),
            pltpu.MemorySpace.VMEM_SHARED((N_PAD, C), jnp.float32),
            pltpu.SemaphoreType.DMA,
            [pltpu.SemaphoreType.DMA for _ in range(NB)],
        ],
    )
    def k(hs_hbm, eidx_hbm, out_hbm, sidx, didx, rows, zbuf, acc, isem, bsem):
        cid = lax.axis_index("c")
        sid = lax.axis_index("s")
        wid = cid * NSUB + sid
        r0 = wid * N_CHUNKS
        sload = pltpu.async_copy(eidx_hbm.at[0, pl.ds(r0, N_CHUNKS)], sidx,
                                 isem)
        dload = pltpu.async_copy(eidx_hbm.at[1, pl.ds(r0, N_CHUNKS)], didx,
                                 isem)
        _zero_fill(zbuf, ZROWS, C)
        row0 = sid * ROWS_PER_TILE

        def zloop(i, carry):
            pltpu.sync_copy(zbuf, acc.at[pl.ds(row0 + i * ZROWS, ZROWS)])
            return carry

        lax.fori_loop(0, ROWS_PER_TILE // ZROWS, zloop, 0)
        sload.wait()
        dload.wait()
        plsc.subcore_barrier()

        def gather(i, b):
            pltpu.async_copy(hs_hbm.at[sidx.at[i]], rows[b], bsem[b])

        def scatter(i, b):
            pltpu.async_copy(rows[b], acc.at[didx.at[i]], bsem[b], add=True)

        def wait_b(b):
            pltpu.make_async_copy(hs_hbm.at[sidx.at[0]], rows[b],
                                  bsem[b]).wait()

        def body(g, carry):
            i0 = g * NB
            for b in range(NB):
                pl.when(g > 0)(lambda b=b: wait_b(b))
                gather(i0 + b, b)
            for b in range(NB):
                wait_b(b)
                scatter(i0 + b, b)
            return carry

        lax.fori_loop(0, N_CHUNKS // NB, body, 0)
        for b in range(NB):
            wait_b(b)
        plsc.subcore_barrier()
        pltpu.sync_copy(
            acc.at[pl.ds(row0, ROWS_PER_TILE)],
            out_hbm.at[pl.ds(row0, ROWS_PER_TILE), pl.ds(cid * C, C)],
        )

    return k


@functools.lru_cache(maxsize=None)
def _make_deg():
    C = 16
    mesh = plsc.VectorSubcoreMesh(core_axis_name="c", subcore_axis_name="s",
                                  num_cores=NCORES, num_subcores=NSUB)

    @functools.partial(
        pl.kernel,
        mesh=mesh,
        out_type=jax.ShapeDtypeStruct((N_PAD, 128), jnp.float32),
        compiler_params=pltpu.CompilerParams(use_tc_tiling_on_sc=False),
        scratch_types=[
            pltpu.VMEM((N_CHUNKS, CHUNK), jnp.int32),
            pltpu.VMEM((CHUNK, C), jnp.float32),
            pltpu.VMEM((ZROWS, C), jnp.float32),
            pltpu.MemorySpace.VMEM_SHARED((N_PAD, C), jnp.float32),
            pltpu.SemaphoreType.DMA,
            [pltpu.SemaphoreType.DMA for _ in range(NBUF)],
        ],
    )
    def k(eidx_hbm, out_hbm, didx, ones, zbuf, acc, isem, bsem):
        cid = lax.axis_index("c")
        sid = lax.axis_index("s")
        wid = cid * NSUB + sid
        r0 = wid * N_CHUNKS
        dload = pltpu.async_copy(eidx_hbm.at[1, pl.ds(r0, N_CHUNKS)], didx,
                                 isem)
        _zero_fill(zbuf, ZROWS, C)
        one = jnp.ones((LANES,), jnp.float32)
        for r in range(CHUNK):
            ones[r, pl.ds(0, LANES)] = one
        row0 = sid * ROWS_PER_TILE

        def zloop(i, carry):
            pltpu.sync_copy(zbuf, acc.at[pl.ds(row0 + i * ZROWS, ZROWS)])
            return carry

        lax.fori_loop(0, ROWS_PER_TILE // ZROWS, zloop, 0)
        dload.wait()
        plsc.subcore_barrier()

        def wait_b(b):
            pltpu.make_async_copy(ones, acc.at[didx.at[0]], bsem[b]).wait()

        def body(g, carry):
            i0 = g * NBUF
            for b in range(NBUF):
                pl.when(g > 0)(lambda b=b: wait_b(b))
                pltpu.async_copy(ones, acc.at[didx.at[i0 + b]], bsem[b],
                                 add=True)
            return carry

        lax.fori_loop(0, N_CHUNKS // NBUF, body, 0)
        for b in range(NBUF):
            wait_b(b)
        plsc.subcore_barrier()
        pltpu.sync_copy(
            acc.at[pl.ds(row0, ROWS_PER_TILE)],
            out_hbm.at[pl.ds(row0, ROWS_PER_TILE), pl.ds(cid * C, C)],
        )

    return k


BN = 2048
GRID = N_PAD // BN


def _r128(n_rows, C):
    return (n_rows * C // 128, 128)


def _tc_mm0(x, W):
    def body(x_ref, w_ref, o_ref):
        o_ref[...] = jnp.dot(x_ref[...], w_ref[...],
                             preferred_element_type=jnp.float32)

    return pl.pallas_call(
        body,
        grid=(GRID,),
        in_specs=[
            pl.BlockSpec((BN, NF), lambda i: (i, 0)),
            pl.BlockSpec((NF, HC1), lambda i: (0, 0)),
        ],
        out_specs=pl.BlockSpec((BN, HC1), lambda i: (i, 0)),
        out_shape=jax.ShapeDtypeStruct((N_PAD, HC1), jnp.float32),
    )(x, W)


def _tc_scale0(h1raw, degp):
    def body(h_ref, d_ref, dis_ref, hs_ref):
        d = d_ref[...]
        dis = lax.rsqrt(1.0 + d[:, 0:1] + d[:, 16:17])
        dis_ref[...] = dis
        hs_ref[...] = h_ref[...] * dis

    return pl.pallas_call(
        body,
        grid=(GRID,),
        in_specs=[
            pl.BlockSpec((BN, HC1), lambda i: (i, 0)),
            pl.BlockSpec((BN, 128), lambda i: (i, 0)),
        ],
        out_specs=[
            pl.BlockSpec((BN, 1), lambda i: (i, 0)),
            pl.BlockSpec((BN, HC1), lambda i: (i, 0)),
        ],
        out_shape=[
            jax.ShapeDtypeStruct((N_PAD, 1), jnp.float32),
            jax.ShapeDtypeStruct((N_PAD, HC1), jnp.float32),
        ],
    )(h1raw, degp)


def _tc_mid(aggp, hs, dis, b, W, C, C2):
    def body(a_ref, hs_ref, dis_ref, b_ref, w_ref, o_ref):
        a = a_ref[...]
        s = a[:, 0:C] + a[:, C:2 * C] + hs_ref[...]
        dis = dis_ref[...]
        t = jnp.tanh(s * dis + b_ref[...])
        o_ref[...] = jnp.dot(t, w_ref[...],
                             preferred_element_type=jnp.float32) * dis

    return pl.pallas_call(
        body,
        grid=(GRID,),
        in_specs=[
            pl.BlockSpec((BN, 128), lambda i: (i, 0)),
            pl.BlockSpec((BN, C), lambda i: (i, 0)),
            pl.BlockSpec((BN, 1), lambda i: (i, 0)),
            pl.BlockSpec((1, C), lambda i: (0, 0)),
            pl.BlockSpec((C, C2), lambda i: (0, 0)),
        ],
        out_specs=pl.BlockSpec((BN, C2), lambda i: (i, 0)),
        out_shape=jax.ShapeDtypeStruct((N_PAD, C2), jnp.float32),
    )(aggp, hs, dis, b, W)


def _tc_emb(aggp, hs, dis, b, W):
    def body(a_ref, hs_ref, dis_ref, b_ref, w_ref, emb_ref, o_ref):
        a = a_ref[...]
        s = a[:, 0:HC2] + a[:, HC2:2 * HC2] + hs_ref[...]
        dis = dis_ref[...]
        emb = s * dis + b_ref[...]
        emb_ref[...] = emb
        o_ref[...] = jnp.dot(jnp.tanh(emb), w_ref[...],
                             preferred_element_type=jnp.float32) * dis

    return pl.pallas_call(
        body,
        grid=(GRID,),
        in_specs=[
            pl.BlockSpec((BN, 128), lambda i: (i, 0)),
            pl.BlockSpec((BN, HC2), lambda i: (i, 0)),
            pl.BlockSpec((BN, 1), lambda i: (i, 0)),
            pl.BlockSpec((1, HC2), lambda i: (0, 0)),
            pl.BlockSpec((HC2, NCLS), lambda i: (0, 0)),
        ],
        out_specs=[
            pl.BlockSpec((BN, HC2), lambda i: (i, 0)),
            pl.BlockSpec((BN, NCLS), lambda i: (i, 0)),
        ],
        out_shape=[
            jax.ShapeDtypeStruct((N_PAD, HC2), jnp.float32),
            jax.ShapeDtypeStruct((N_PAD, NCLS), jnp.float32),
        ],
    )(aggp, hs, dis, b, W)


def _tc_final(aggp, hs, dis, b):
    def body(a_ref, hs_ref, dis_ref, b_ref, o_ref):
        a = a_ref[...]
        s = a[:, 0:NCLS] + a[:, NCLS:2 * NCLS] + hs_ref[...]
        o_ref[...] = s * dis_ref[...] + b_ref[...]

    return pl.pallas_call(
        body,
        grid=(GRID,),
        in_specs=[
            pl.BlockSpec((BN, 128), lambda i: (i, 0)),
            pl.BlockSpec((BN, NCLS), lambda i: (i, 0)),
            pl.BlockSpec((BN, 1), lambda i: (i, 0)),
            pl.BlockSpec((1, NCLS), lambda i: (0, 0)),
        ],
        out_specs=pl.BlockSpec((BN, NCLS), lambda i: (i, 0)),
        out_shape=jax.ShapeDtypeStruct((N_PAD, NCLS), jnp.float32),
    )(aggp, hs, dis, b)


def _v128(a, C):
    return a.reshape(NCORES, N_PAD * C // 128, 128)


def kernel(x, edge_index, W1, b1, W2, b2, W3, b3):
    eidx = edge_index.reshape(2, NW * N_CHUNKS, CHUNK)
    x_p = jnp.pad(x, ((0, N_PAD - N), (0, 0)))

    degp = _make_deg()(eidx)
    h1raw = _tc_mm0(x_p, W1)
    dis, hs1 = _tc_scale0(h1raw, degp)

    aggp1 = _make_agg(HC1)(hs1, eidx)
    hs2 = _tc_mid(aggp1, hs1, dis, b1.reshape(1, HC1), W2, HC1, HC2)

    aggp2 = _make_agg(HC2)(hs2, eidx)
    emb_p, hs3 = _tc_emb(aggp2, hs2, dis, b2.reshape(1, HC2), W3)

    aggp3 = _make_agg(NCLS)(hs3, eidx)
    logits_p = _tc_final(aggp3, hs3, dis, b3.reshape(1, NCLS))

    return (logits_p[:N], emb_p[:N])


# speedup vs baseline: 55.0817x; 1.0138x over previous
import functools

import jax
import jax.numpy as jnp
from jax import lax
from jax.experimental import pallas as pl
from jax.experimental.pallas import tpu as pltpu
from jax.experimental.pallas import tpu_sc as plsc

N = 10000
E = 320000
NF = 128
HC1 = 64
HC2 = 32
NCLS = 16

NCORES = 2
NSUB = 16
NW = NCORES * NSUB
LANES = 16

CHUNK = 125
NBUF = 5
N_CHUNKS = E // (NW * CHUNK)
N_PAD = 10240
ROWS_PER_TILE = N_PAD // NSUB
ZROWS = 128

assert NW * N_CHUNKS * CHUNK == E


def _zero_fill(buf, rows, cols):
    zero = jnp.zeros((LANES,), jnp.float32)
    for r in range(rows):
        for c in range(cols // LANES):
            buf[r, pl.ds(c * LANES, LANES)] = zero


@functools.lru_cache(maxsize=None)
def _make_agg(C):
    NB = NBUF if C >= 64 else 10
    mesh = plsc.VectorSubcoreMesh(core_axis_name="c", subcore_axis_name="s",
                                  num_cores=NCORES, num_subcores=NSUB)

    @functools.partial(
        pl.kernel,
        mesh=mesh,
        out_type=jax.ShapeDtypeStruct((N_PAD, 128), jnp.float32),
        compiler_params=pltpu.CompilerParams(use_tc_tiling_on_sc=False),
        scratch_types=[
            pltpu.VMEM((N_CHUNKS, CHUNK), jnp.int32),
            pltpu.VMEM((N_CHUNKS, CHUNK), jnp.int32),
            [pltpu.VMEM((CHUNK, C), jnp.float32) for _ in range(NB)],
            pltpu.VMEM((ZROWS, C), jnp.float32),
            pltpu.MemorySpace.VMEM_SHARED((N_PAD, C), jnp.float32),
            pltpu.SemaphoreType.DMA,
            [pltpu.SemaphoreType.DMA for _ in range(NB)],
        ],
    )
    def k(hs_hbm, eidx_hbm, out_hbm, sidx, didx, rows, zbuf, acc, isem, bsem):
        cid = lax.axis_index("c")
        sid = lax.axis_index("s")
        wid = cid * NSUB + sid
        r0 = wid * N_CHUNKS
        sload = pltpu.async_copy(eidx_hbm.at[0, pl.ds(r0, N_CHUNKS)], sidx,
                                 isem)
        dload = pltpu.async_copy(eidx_hbm.at[1, pl.ds(r0, N_CHUNKS)], didx,
                                 isem)
        _zero_fill(zbuf, ZROWS, C)
        row0 = sid * ROWS_PER_TILE

        def zloop(i, carry):
            pltpu.sync_copy(zbuf, acc.at[pl.ds(row0 + i * ZROWS, ZROWS)])
            return carry

        lax.fori_loop(0, ROWS_PER_TILE // ZROWS, zloop, 0)
        sload.wait()
        dload.wait()
        plsc.subcore_barrier()

        def gather(i, b):
            pltpu.async_copy(hs_hbm.at[sidx.at[i]], rows[b], bsem[b])

        def scatter(i, b):
            pltpu.async_copy(rows[b], acc.at[didx.at[i]], bsem[b], add=True)

        def wait_b(b):
            pltpu.make_async_copy(hs_hbm.at[sidx.at[0]], rows[b],
                                  bsem[b]).wait()

        def body(g, carry):
            i0 = g * NB
            for b in range(NB):
                pl.when(g > 0)(lambda b=b: wait_b(b))
                gather(i0 + b, b)
            for b in range(NB):
                wait_b(b)
                scatter(i0 + b, b)
            return carry

        lax.fori_loop(0, N_CHUNKS // NB, body, 0)
        for b in range(NB):
            wait_b(b)
        plsc.subcore_barrier()
        pltpu.sync_copy(
            acc.at[pl.ds(row0, ROWS_PER_TILE)],
            out_hbm.at[pl.ds(row0, ROWS_PER_TILE), pl.ds(cid * C, C)],
        )

    return k


@functools.lru_cache(maxsize=None)
def _make_deg():
    C = 16
    mesh = plsc.VectorSubcoreMesh(core_axis_name="c", subcore_axis_name="s",
                                  num_cores=NCORES, num_subcores=NSUB)

    @functools.partial(
        pl.kernel,
        mesh=mesh,
        out_type=jax.ShapeDtypeStruct((N_PAD, 128), jnp.float32),
        compiler_params=pltpu.CompilerParams(use_tc_tiling_on_sc=False),
        scratch_types=[
            pltpu.VMEM((N_CHUNKS, CHUNK), jnp.int32),
            pltpu.VMEM((CHUNK, C), jnp.float32),
            pltpu.VMEM((ZROWS, C), jnp.float32),
            pltpu.MemorySpace.VMEM_SHARED((N_PAD, C), jnp.float32),
            pltpu.SemaphoreType.DMA,
            [pltpu.SemaphoreType.DMA for _ in range(NBUF)],
        ],
    )
    def k(eidx_hbm, out_hbm, didx, ones, zbuf, acc, isem, bsem):
        cid = lax.axis_index("c")
        sid = lax.axis_index("s")
        wid = cid * NSUB + sid
        r0 = wid * N_CHUNKS
        dload = pltpu.async_copy(eidx_hbm.at[1, pl.ds(r0, N_CHUNKS)], didx,
                                 isem)
        _zero_fill(zbuf, ZROWS, C)
        one = jnp.ones((LANES,), jnp.float32)
        for r in range(CHUNK):
            ones[r, pl.ds(0, LANES)] = one
        row0 = sid * ROWS_PER_TILE

        def zloop(i, carry):
            pltpu.sync_copy(zbuf, acc.at[pl.ds(row0 + i * ZROWS, ZROWS)])
            return carry

        lax.fori_loop(0, ROWS_PER_TILE // ZROWS, zloop, 0)
        dload.wait()
        plsc.subcore_barrier()

        def wait_b(b):
            pltpu.make_async_copy(ones, acc.at[didx.at[0]], bsem[b]).wait()

        def body(g, carry):
            i0 = g * NBUF
            for b in range(NBUF):
                pl.when(g > 0)(lambda b=b: wait_b(b))
                pltpu.async_copy(ones, acc.at[didx.at[i0 + b]], bsem[b],
                                 add=True)
            return carry

        lax.fori_loop(0, N_CHUNKS // NBUF, body, 0)
        for b in range(NBUF):
            wait_b(b)
        plsc.subcore_barrier()
        pltpu.sync_copy(
            acc.at[pl.ds(row0, ROWS_PER_TILE)],
            out_hbm.at[pl.ds(row0, ROWS_PER_TILE), pl.ds(cid * C, C)],
        )

    return k


BN = 2048
GRID = N_PAD // BN


def _r128(n_rows, C):
    return (n_rows * C // 128, 128)


def _tc_mm0(x, W):
    def body(x_ref, w_ref, o_ref):
        o_ref[...] = jnp.dot(x_ref[...], w_ref[...],
                             preferred_element_type=jnp.float32)

    return pl.pallas_call(
        body,
        grid=(GRID,),
        in_specs=[
            pl.BlockSpec((BN, NF), lambda i: (i, 0)),
            pl.BlockSpec((NF, HC1), lambda i: (0, 0)),
        ],
        out_specs=pl.BlockSpec((BN, HC1), lambda i: (i, 0)),
        out_shape=jax.ShapeDtypeStruct((N_PAD, HC1), jnp.float32),
    )(x, W)


def _tc_scale0(h1raw, degp):
    def body(h_ref, d_ref, dis_ref, hs_ref):
        d = d_ref[...]
        dis = lax.rsqrt(1.0 + d[:, 0:1] + d[:, 16:17])
        dis_ref[...] = dis
        hs_ref[...] = h_ref[...] * dis

    return pl.pallas_call(
        body,
        grid=(GRID,),
        in_specs=[
            pl.BlockSpec((BN, HC1), lambda i: (i, 0)),
            pl.BlockSpec((BN, 128), lambda i: (i, 0)),
        ],
        out_specs=[
            pl.BlockSpec((BN, 1), lambda i: (i, 0)),
            pl.BlockSpec((BN, HC1), lambda i: (i, 0)),
        ],
        out_shape=[
            jax.ShapeDtypeStruct((N_PAD, 1), jnp.float32),
            jax.ShapeDtypeStruct((N_PAD, HC1), jnp.float32),
        ],
    )(h1raw, degp)


def _tc_mid(aggp, hs, dis, b, W, C, C2):
    def body(a_ref, hs_ref, dis_ref, b_ref, w_ref, o_ref):
        a = a_ref[...]
        s = a[:, 0:C] + a[:, C:2 * C] + hs_ref[...]
        dis = dis_ref[...]
        t = jnp.tanh(s * dis + b_ref[...])
        o_ref[...] = jnp.dot(t, w_ref[...],
                             preferred_element_type=jnp.float32) * dis

    return pl.pallas_call(
        body,
        grid=(GRID,),
        in_specs=[
            pl.BlockSpec((BN, 128), lambda i: (i, 0)),
            pl.BlockSpec((BN, C), lambda i: (i, 0)),
            pl.BlockSpec((BN, 1), lambda i: (i, 0)),
            pl.BlockSpec((1, C), lambda i: (0, 0)),
            pl.BlockSpec((C, C2), lambda i: (0, 0)),
        ],
        out_specs=pl.BlockSpec((BN, C2), lambda i: (i, 0)),
        out_shape=jax.ShapeDtypeStruct((N_PAD, C2), jnp.float32),
    )(aggp, hs, dis, b, W)


def _tc_emb(aggp, hs, dis, b, W):
    def body(a_ref, hs_ref, dis_ref, b_ref, w_ref, emb_ref, o_ref):
        a = a_ref[...]
        s = a[:, 0:HC2] + a[:, HC2:2 * HC2] + hs_ref[...]
        dis = dis_ref[...]
        emb = s * dis + b_ref[...]
        emb_ref[...] = emb
        o_ref[...] = jnp.dot(jnp.tanh(emb), w_ref[...],
                             preferred_element_type=jnp.float32) * dis

    return pl.pallas_call(
        body,
        grid=(GRID,),
        in_specs=[
            pl.BlockSpec((BN, 128), lambda i: (i, 0)),
            pl.BlockSpec((BN, HC2), lambda i: (i, 0)),
            pl.BlockSpec((BN, 1), lambda i: (i, 0)),
            pl.BlockSpec((1, HC2), lambda i: (0, 0)),
            pl.BlockSpec((HC2, NCLS), lambda i: (0, 0)),
        ],
        out_specs=[
            pl.BlockSpec((BN, HC2), lambda i: (i, 0)),
            pl.BlockSpec((BN, NCLS), lambda i: (i, 0)),
        ],
        out_shape=[
            jax.ShapeDtypeStruct((N, HC2), jnp.float32),
            jax.ShapeDtypeStruct((N_PAD, NCLS), jnp.float32),
        ],
    )(aggp, hs, dis, b, W)


def _tc_final(aggp, hs, dis, b):
    def body(a_ref, hs_ref, dis_ref, b_ref, o_ref):
        a = a_ref[...]
        s = a[:, 0:NCLS] + a[:, NCLS:2 * NCLS] + hs_ref[...]
        o_ref[...] = s * dis_ref[...] + b_ref[...]

    return pl.pallas_call(
        body,
        grid=(GRID,),
        in_specs=[
            pl.BlockSpec((BN, 128), lambda i: (i, 0)),
            pl.BlockSpec((BN, NCLS), lambda i: (i, 0)),
            pl.BlockSpec((BN, 1), lambda i: (i, 0)),
            pl.BlockSpec((1, NCLS), lambda i: (0, 0)),
        ],
        out_specs=pl.BlockSpec((BN, NCLS), lambda i: (i, 0)),
        out_shape=jax.ShapeDtypeStruct((N, NCLS), jnp.float32),
    )(aggp, hs, dis, b)


def _v128(a, C):
    return a.reshape(NCORES, N_PAD * C // 128, 128)


def kernel(x, edge_index, W1, b1, W2, b2, W3, b3):
    eidx = edge_index.reshape(2, NW * N_CHUNKS, CHUNK)

    degp = _make_deg()(eidx)
    h1raw = _tc_mm0(x, W1)
    dis, hs1 = _tc_scale0(h1raw, degp)

    aggp1 = _make_agg(HC1)(hs1, eidx)
    hs2 = _tc_mid(aggp1, hs1, dis, b1.reshape(1, HC1), W2, HC1, HC2)

    aggp2 = _make_agg(HC2)(hs2, eidx)
    emb_p, hs3 = _tc_emb(aggp2, hs2, dis, b2.reshape(1, HC2), W3)

    aggp3 = _make_agg(NCLS)(hs3, eidx)
    logits = _tc_final(aggp3, hs3, dis, b3.reshape(1, NCLS))

    return (logits, emb_p)


# speedup vs baseline: 55.8446x; 1.0139x over previous
import functools

import jax
import jax.numpy as jnp
from jax import lax
from jax.experimental import pallas as pl
from jax.experimental.pallas import tpu as pltpu
from jax.experimental.pallas import tpu_sc as plsc

N = 10000
E = 320000
NF = 128
HC1 = 64
HC2 = 32
NCLS = 16

NCORES = 2
NSUB = 16
NW = NCORES * NSUB
LANES = 16

CHUNK = 125
NBUF = 5
N_CHUNKS = E // (NW * CHUNK)
N_PAD = 10240
ROWS_PER_TILE = N_PAD // NSUB
ZROWS = 128

assert NW * N_CHUNKS * CHUNK == E


def _zero_fill(buf, rows, cols):
    zero = jnp.zeros((LANES,), jnp.float32)
    for r in range(rows):
        for c in range(cols // LANES):
            buf[r, pl.ds(c * LANES, LANES)] = zero


@functools.lru_cache(maxsize=None)
def _make_agg(C):
    NB = NBUF if C >= 64 else 16
    mesh = plsc.VectorSubcoreMesh(core_axis_name="c", subcore_axis_name="s",
                                  num_cores=NCORES, num_subcores=NSUB)

    @functools.partial(
        pl.kernel,
        mesh=mesh,
        out_type=jax.ShapeDtypeStruct((N_PAD, 128), jnp.float32),
        compiler_params=pltpu.CompilerParams(use_tc_tiling_on_sc=False),
        scratch_types=[
            pltpu.VMEM((N_CHUNKS, CHUNK), jnp.int32),
            pltpu.VMEM((N_CHUNKS, CHUNK), jnp.int32),
            [pltpu.VMEM((CHUNK, C), jnp.float32) for _ in range(NB)],
            pltpu.VMEM((ZROWS, C), jnp.float32),
            pltpu.MemorySpace.VMEM_SHARED((N_PAD, C), jnp.float32),
            pltpu.SemaphoreType.DMA,
            [pltpu.SemaphoreType.DMA for _ in range(NB)],
        ],
    )
    def k(hs_hbm, eidx_hbm, out_hbm, sidx, didx, rows, zbuf, acc, isem, bsem):
        cid = lax.axis_index("c")
        sid = lax.axis_index("s")
        wid = cid * NSUB + sid
        r0 = wid * N_CHUNKS
        sload = pltpu.async_copy(eidx_hbm.at[0, pl.ds(r0, N_CHUNKS)], sidx,
                                 isem)
        dload = pltpu.async_copy(eidx_hbm.at[1, pl.ds(r0, N_CHUNKS)], didx,
                                 isem)
        _zero_fill(zbuf, ZROWS, C)
        row0 = sid * ROWS_PER_TILE

        def zloop(i, carry):
            pltpu.sync_copy(zbuf, acc.at[pl.ds(row0 + i * ZROWS, ZROWS)])
            return carry

        lax.fori_loop(0, ROWS_PER_TILE // ZROWS, zloop, 0)
        sload.wait()
        dload.wait()
        plsc.subcore_barrier()

        def gather(i, b):
            pltpu.async_copy(hs_hbm.at[sidx.at[i]], rows[b], bsem[b])

        def scatter(i, b):
            pltpu.async_copy(rows[b], acc.at[didx.at[i]], bsem[b], add=True)

        def wait_b(b):
            pltpu.make_async_copy(hs_hbm.at[sidx.at[0]], rows[b],
                                  bsem[b]).wait()

        def body(g, carry):
            i0 = g * NB
            for b in range(NB):
                pl.when(g > 0)(lambda b=b: wait_b(b))
                gather(i0 + b, b)
            for b in range(NB):
                wait_b(b)
                scatter(i0 + b, b)
            return carry

        lax.fori_loop(0, N_CHUNKS // NB, body, 0)
        for b in range(NB):
            wait_b(b)
        plsc.subcore_barrier()
        pltpu.sync_copy(
            acc.at[pl.ds(row0, ROWS_PER_TILE)],
            out_hbm.at[pl.ds(row0, ROWS_PER_TILE), pl.ds(cid * C, C)],
        )

    return k


@functools.lru_cache(maxsize=None)
def _make_deg():
    C = 16
    mesh = plsc.VectorSubcoreMesh(core_axis_name="c", subcore_axis_name="s",
                                  num_cores=NCORES, num_subcores=NSUB)

    @functools.partial(
        pl.kernel,
        mesh=mesh,
        out_type=jax.ShapeDtypeStruct((N_PAD, 128), jnp.float32),
        compiler_params=pltpu.CompilerParams(use_tc_tiling_on_sc=False),
        scratch_types=[
            pltpu.VMEM((N_CHUNKS, CHUNK), jnp.int32),
            pltpu.VMEM((CHUNK, C), jnp.float32),
            pltpu.VMEM((ZROWS, C), jnp.float32),
            pltpu.MemorySpace.VMEM_SHARED((N_PAD, C), jnp.float32),
            pltpu.SemaphoreType.DMA,
            [pltpu.SemaphoreType.DMA for _ in range(NBUF)],
        ],
    )
    def k(eidx_hbm, out_hbm, didx, ones, zbuf, acc, isem, bsem):
        cid = lax.axis_index("c")
        sid = lax.axis_index("s")
        wid = cid * NSUB + sid
        r0 = wid * N_CHUNKS
        dload = pltpu.async_copy(eidx_hbm.at[1, pl.ds(r0, N_CHUNKS)], didx,
                                 isem)
        _zero_fill(zbuf, ZROWS, C)
        one = jnp.ones((LANES,), jnp.float32)
        for r in range(CHUNK):
            ones[r, pl.ds(0, LANES)] = one
        row0 = sid * ROWS_PER_TILE

        def zloop(i, carry):
            pltpu.sync_copy(zbuf, acc.at[pl.ds(row0 + i * ZROWS, ZROWS)])
            return carry

        lax.fori_loop(0, ROWS_PER_TILE // ZROWS, zloop, 0)
        dload.wait()
        plsc.subcore_barrier()

        def wait_b(b):
            pltpu.make_async_copy(ones, acc.at[didx.at[0]], bsem[b]).wait()

        def body(g, carry):
            i0 = g * NBUF
            for b in range(NBUF):
                pl.when(g > 0)(lambda b=b: wait_b(b))
                pltpu.async_copy(ones, acc.at[didx.at[i0 + b]], bsem[b],
                                 add=True)
            return carry

        lax.fori_loop(0, N_CHUNKS // NBUF, body, 0)
        for b in range(NBUF):
            wait_b(b)
        plsc.subcore_barrier()
        pltpu.sync_copy(
            acc.at[pl.ds(row0, ROWS_PER_TILE)],
            out_hbm.at[pl.ds(row0, ROWS_PER_TILE), pl.ds(cid * C, C)],
        )

    return k


BN = 2048
GRID = N_PAD // BN


def _r128(n_rows, C):
    return (n_rows * C // 128, 128)


def _tc_mm0(x, W):
    def body(x_ref, w_ref, o_ref):
        o_ref[...] = jnp.dot(x_ref[...], w_ref[...],
                             preferred_element_type=jnp.float32)

    return pl.pallas_call(
        body,
        grid=(GRID,),
        in_specs=[
            pl.BlockSpec((BN, NF), lambda i: (i, 0)),
            pl.BlockSpec((NF, HC1), lambda i: (0, 0)),
        ],
        out_specs=pl.BlockSpec((BN, HC1), lambda i: (i, 0)),
        out_shape=jax.ShapeDtypeStruct((N_PAD, HC1), jnp.float32),
    )(x, W)


def _tc_scale0(h1raw, degp):
    def body(h_ref, d_ref, dis_ref, hs_ref):
        d = d_ref[...]
        dis = lax.rsqrt(1.0 + d[:, 0:1] + d[:, 16:17])
        dis_ref[...] = dis
        hs_ref[...] = h_ref[...] * dis

    return pl.pallas_call(
        body,
        grid=(GRID,),
        in_specs=[
            pl.BlockSpec((BN, HC1), lambda i: (i, 0)),
            pl.BlockSpec((BN, 128), lambda i: (i, 0)),
        ],
        out_specs=[
            pl.BlockSpec((BN, 1), lambda i: (i, 0)),
            pl.BlockSpec((BN, HC1), lambda i: (i, 0)),
        ],
        out_shape=[
            jax.ShapeDtypeStruct((N_PAD, 1), jnp.float32),
            jax.ShapeDtypeStruct((N_PAD, HC1), jnp.float32),
        ],
    )(h1raw, degp)


def _tc_mid(aggp, hs, dis, b, W, C, C2):
    def body(a_ref, hs_ref, dis_ref, b_ref, w_ref, o_ref):
        a = a_ref[...]
        s = a[:, 0:C] + a[:, C:2 * C] + hs_ref[...]
        dis = dis_ref[...]
        t = jnp.tanh(s * dis + b_ref[...])
        o_ref[...] = jnp.dot(t, w_ref[...],
                             preferred_element_type=jnp.float32) * dis

    return pl.pallas_call(
        body,
        grid=(GRID,),
        in_specs=[
            pl.BlockSpec((BN, 128), lambda i: (i, 0)),
            pl.BlockSpec((BN, C), lambda i: (i, 0)),
            pl.BlockSpec((BN, 1), lambda i: (i, 0)),
            pl.BlockSpec((1, C), lambda i: (0, 0)),
            pl.BlockSpec((C, C2), lambda i: (0, 0)),
        ],
        out_specs=pl.BlockSpec((BN, C2), lambda i: (i, 0)),
        out_shape=jax.ShapeDtypeStruct((N_PAD, C2), jnp.float32),
    )(aggp, hs, dis, b, W)


def _tc_emb(aggp, hs, dis, b, W):
    def body(a_ref, hs_ref, dis_ref, b_ref, w_ref, emb_ref, o_ref):
        a = a_ref[...]
        s = a[:, 0:HC2] + a[:, HC2:2 * HC2] + hs_ref[...]
        dis = dis_ref[...]
        emb = s * dis + b_ref[...]
        emb_ref[...] = emb
        o_ref[...] = jnp.dot(jnp.tanh(emb), w_ref[...],
                             preferred_element_type=jnp.float32) * dis

    return pl.pallas_call(
        body,
        grid=(GRID,),
        in_specs=[
            pl.BlockSpec((BN, 128), lambda i: (i, 0)),
            pl.BlockSpec((BN, HC2), lambda i: (i, 0)),
            pl.BlockSpec((BN, 1), lambda i: (i, 0)),
            pl.BlockSpec((1, HC2), lambda i: (0, 0)),
            pl.BlockSpec((HC2, NCLS), lambda i: (0, 0)),
        ],
        out_specs=[
            pl.BlockSpec((BN, HC2), lambda i: (i, 0)),
            pl.BlockSpec((BN, NCLS), lambda i: (i, 0)),
        ],
        out_shape=[
            jax.ShapeDtypeStruct((N, HC2), jnp.float32),
            jax.ShapeDtypeStruct((N_PAD, NCLS), jnp.float32),
        ],
    )(aggp, hs, dis, b, W)


def _tc_final(aggp, hs, dis, b):
    def body(a_ref, hs_ref, dis_ref, b_ref, o_ref):
        a = a_ref[...]
        s = a[:, 0:NCLS] + a[:, NCLS:2 * NCLS] + hs_ref[...]
        o_ref[...] = s * dis_ref[...] + b_ref[...]

    return pl.pallas_call(
        body,
        grid=(GRID,),
        in_specs=[
            pl.BlockSpec((BN, 128), lambda i: (i, 0)),
            pl.BlockSpec((BN, NCLS), lambda i: (i, 0)),
            pl.BlockSpec((BN, 1), lambda i: (i, 0)),
            pl.BlockSpec((1, NCLS), lambda i: (0, 0)),
        ],
        out_specs=pl.BlockSpec((BN, NCLS), lambda i: (i, 0)),
        out_shape=jax.ShapeDtypeStruct((N, NCLS), jnp.float32),
    )(aggp, hs, dis, b)


def _v128(a, C):
    return a.reshape(NCORES, N_PAD * C // 128, 128)


def kernel(x, edge_index, W1, b1, W2, b2, W3, b3):
    eidx = edge_index.reshape(2, NW * N_CHUNKS, CHUNK)

    degp = _make_deg()(eidx)
    h1raw = _tc_mm0(x, W1)
    dis, hs1 = _tc_scale0(h1raw, degp)

    aggp1 = _make_agg(HC1)(hs1, eidx)
    hs2 = _tc_mid(aggp1, hs1, dis, b1.reshape(1, HC1), W2, HC1, HC2)

    aggp2 = _make_agg(HC2)(hs2, eidx)
    emb_p, hs3 = _tc_emb(aggp2, hs2, dis, b2.reshape(1, HC2), W3)

    aggp3 = _make_agg(NCLS)(hs3, eidx)
    logits = _tc_final(aggp3, hs3, dis, b3.reshape(1, NCLS))

    return (logits, emb_p)


# speedup vs baseline: 57.0397x; 1.0214x over previous
import functools

import jax
import jax.numpy as jnp
from jax import lax
from jax.experimental import pallas as pl
from jax.experimental.pallas import tpu as pltpu
from jax.experimental.pallas import tpu_sc as plsc

N = 10000
E = 320000
NF = 128
HC1 = 64
HC2 = 32
NCLS = 16

NCORES = 2
NSUB = 16
NW = NCORES * NSUB
LANES = 16

CHUNK = 125
NBUF = 5
N_CHUNKS = E // (NW * CHUNK)
N_PAD = 10240
ROWS_PER_TILE = N_PAD // NSUB
ZROWS = 128

assert NW * N_CHUNKS * CHUNK == E


def _zero_fill(buf, rows, cols):
    zero = jnp.zeros((LANES,), jnp.float32)
    for r in range(rows):
        for c in range(cols // LANES):
            buf[r, pl.ds(c * LANES, LANES)] = zero


@functools.lru_cache(maxsize=None)
def _make_agg(C):
    NB = NBUF if C >= 64 else 16
    mesh = plsc.VectorSubcoreMesh(core_axis_name="c", subcore_axis_name="s",
                                  num_cores=NCORES, num_subcores=NSUB)

    @functools.partial(
        pl.kernel,
        mesh=mesh,
        out_type=jax.ShapeDtypeStruct((N_PAD, 128), jnp.float32),
        compiler_params=pltpu.CompilerParams(use_tc_tiling_on_sc=False),
        scratch_types=[
            pltpu.VMEM((N_CHUNKS, CHUNK), jnp.int32),
            pltpu.VMEM((N_CHUNKS, CHUNK), jnp.int32),
            [pltpu.VMEM((CHUNK, C), jnp.float32) for _ in range(NB)],
            pltpu.VMEM((ZROWS, C), jnp.float32),
            pltpu.MemorySpace.VMEM_SHARED((N_PAD, C), jnp.float32),
            pltpu.SemaphoreType.DMA,
            [pltpu.SemaphoreType.DMA for _ in range(NB)],
        ],
    )
    def k(hs_hbm, eidx_hbm, out_hbm, sidx, didx, rows, zbuf, acc, isem, bsem):
        cid = lax.axis_index("c")
        sid = lax.axis_index("s")
        wid = cid * NSUB + sid
        r0 = wid * N_CHUNKS
        sload = pltpu.async_copy(eidx_hbm.at[0, pl.ds(r0, N_CHUNKS)], sidx,
                                 isem)
        dload = pltpu.async_copy(eidx_hbm.at[1, pl.ds(r0, N_CHUNKS)], didx,
                                 isem)
        _zero_fill(zbuf, ZROWS, C)
        row0 = sid * ROWS_PER_TILE

        def zloop(i, carry):
            pltpu.sync_copy(zbuf, acc.at[pl.ds(row0 + i * ZROWS, ZROWS)])
            return carry

        lax.fori_loop(0, ROWS_PER_TILE // ZROWS, zloop, 0)
        sload.wait()
        dload.wait()
        plsc.subcore_barrier()

        def gather(i, b):
            pltpu.async_copy(hs_hbm.at[sidx.at[i]], rows[b], bsem[b])

        def scatter(i, b):
            pltpu.async_copy(rows[b], acc.at[didx.at[i]], bsem[b], add=True)

        def wait_b(b):
            pltpu.make_async_copy(hs_hbm.at[sidx.at[0]], rows[b],
                                  bsem[b]).wait()

        def body(g, carry):
            i0 = g * NB
            for b in range(NB):
                pl.when(g > 0)(lambda b=b: wait_b(b))
                gather(i0 + b, b)
            for b in range(NB):
                wait_b(b)
                scatter(i0 + b, b)
            return carry

        lax.fori_loop(0, N_CHUNKS // NB, body, 0)
        for b in range(NB):
            wait_b(b)
        plsc.subcore_barrier()
        pltpu.sync_copy(
            acc.at[pl.ds(row0, ROWS_PER_TILE)],
            out_hbm.at[pl.ds(row0, ROWS_PER_TILE), pl.ds(cid * C, C)],
        )

    return k


@functools.lru_cache(maxsize=None)
def _make_deg():
    C = 16
    NTAIL = (CHUNK // 16) * 16 - (CHUNK - 16)
    mesh = plsc.VectorSubcoreMesh(core_axis_name="c", subcore_axis_name="s",
                                  num_cores=NCORES, num_subcores=NSUB)

    @functools.partial(
        pl.kernel,
        mesh=mesh,
        out_type=jax.ShapeDtypeStruct((N_PAD, 128), jnp.float32),
        compiler_params=pltpu.CompilerParams(use_tc_tiling_on_sc=False,
                                             needs_layout_passes=False),
        scratch_types=[
            pltpu.VMEM((N_CHUNKS, CHUNK), jnp.int32),
            pltpu.VMEM((N_PAD,), jnp.float32),
            pltpu.VMEM((NSUB, ROWS_PER_TILE), jnp.float32),
            pltpu.VMEM((ROWS_PER_TILE, C), jnp.float32),
            pltpu.MemorySpace.VMEM_SHARED((NSUB, N_PAD), jnp.float32),
            pltpu.SemaphoreType.DMA,
        ],
    )
    def k(eidx_hbm, out_hbm, didx, hist, parts, res, shared, isem):
        cid = lax.axis_index("c")
        sid = lax.axis_index("s")
        wid = cid * NSUB + sid
        r0 = wid * N_CHUNKS
        dload = pltpu.async_copy(eidx_hbm.at[1, pl.ds(r0, N_CHUNKS)], didx,
                                 isem)
        zero = jnp.zeros((LANES,), jnp.float32)

        def zloop(i, carry):
            for c in range(16):
                hist[pl.ds(i * 256 + c * 16, 16)] = zero
            return carry

        lax.fori_loop(0, N_PAD // 256, zloop, 0)
        dload.wait()

        ones = jnp.ones((LANES,), jnp.float32)
        lane = lax.iota(jnp.int32, 16)

        def hrow(r, carry):
            for c in range(CHUNK // 16):
                plsc.addupdate_scatter(hist, [didx[r, pl.ds(c * 16, 16)]],
                                       ones)
            plsc.addupdate_scatter(hist, [didx[r, pl.ds(CHUNK - 16, 16)]],
                                   ones, mask=lane >= NTAIL)
            return carry

        lax.fori_loop(0, N_CHUNKS, hrow, 0)
        pltpu.sync_copy(hist, shared.at[sid])
        plsc.subcore_barrier()

        row0 = sid * ROWS_PER_TILE
        pltpu.sync_copy(shared.at[:, pl.ds(row0, ROWS_PER_TILE)], parts)
        col0 = lane * 0

        def cloop(j, carry):
            tot = parts[0, pl.ds(j * 16, 16)]
            for t in range(1, NSUB):
                tot = tot + parts[t, pl.ds(j * 16, 16)]
            plsc.store_scatter(res, [j * 16 + lane, col0], tot)
            return carry

        lax.fori_loop(0, ROWS_PER_TILE // 16, cloop, 0)
        pltpu.sync_copy(
            res,
            out_hbm.at[pl.ds(row0, ROWS_PER_TILE), pl.ds(cid * C, C)],
        )

    return k


BN = 2048
GRID = N_PAD // BN


def _r128(n_rows, C):
    return (n_rows * C // 128, 128)


def _tc_mm0(x, W):
    def body(x_ref, w_ref, o_ref):
        o_ref[...] = jnp.dot(x_ref[...], w_ref[...],
                             preferred_element_type=jnp.float32)

    return pl.pallas_call(
        body,
        grid=(GRID,),
        in_specs=[
            pl.BlockSpec((BN, NF), lambda i: (i, 0)),
            pl.BlockSpec((NF, HC1), lambda i: (0, 0)),
        ],
        out_specs=pl.BlockSpec((BN, HC1), lambda i: (i, 0)),
        out_shape=jax.ShapeDtypeStruct((N_PAD, HC1), jnp.float32),
    )(x, W)


def _tc_scale0(h1raw, degp):
    def body(h_ref, d_ref, dis_ref, hs_ref):
        d = d_ref[...]
        dis = lax.rsqrt(1.0 + d[:, 0:1] + d[:, 16:17])
        dis_ref[...] = dis
        hs_ref[...] = h_ref[...] * dis

    return pl.pallas_call(
        body,
        grid=(GRID,),
        in_specs=[
            pl.BlockSpec((BN, HC1), lambda i: (i, 0)),
            pl.BlockSpec((BN, 128), lambda i: (i, 0)),
        ],
        out_specs=[
            pl.BlockSpec((BN, 1), lambda i: (i, 0)),
            pl.BlockSpec((BN, HC1), lambda i: (i, 0)),
        ],
        out_shape=[
            jax.ShapeDtypeStruct((N_PAD, 1), jnp.float32),
            jax.ShapeDtypeStruct((N_PAD, HC1), jnp.float32),
        ],
    )(h1raw, degp)


def _tc_mid(aggp, hs, dis, b, W, C, C2):
    def body(a_ref, hs_ref, dis_ref, b_ref, w_ref, o_ref):
        a = a_ref[...]
        s = a[:, 0:C] + a[:, C:2 * C] + hs_ref[...]
        dis = dis_ref[...]
        t = jnp.tanh(s * dis + b_ref[...])
        o_ref[...] = jnp.dot(t, w_ref[...],
                             preferred_element_type=jnp.float32) * dis

    return pl.pallas_call(
        body,
        grid=(GRID,),
        in_specs=[
            pl.BlockSpec((BN, 128), lambda i: (i, 0)),
            pl.BlockSpec((BN, C), lambda i: (i, 0)),
            pl.BlockSpec((BN, 1), lambda i: (i, 0)),
            pl.BlockSpec((1, C), lambda i: (0, 0)),
            pl.BlockSpec((C, C2), lambda i: (0, 0)),
        ],
        out_specs=pl.BlockSpec((BN, C2), lambda i: (i, 0)),
        out_shape=jax.ShapeDtypeStruct((N_PAD, C2), jnp.float32),
    )(aggp, hs, dis, b, W)


def _tc_emb(aggp, hs, dis, b, W):
    def body(a_ref, hs_ref, dis_ref, b_ref, w_ref, emb_ref, o_ref):
        a = a_ref[...]
        s = a[:, 0:HC2] + a[:, HC2:2 * HC2] + hs_ref[...]
        dis = dis_ref[...]
        emb = s * dis + b_ref[...]
        emb_ref[...] = emb
        o_ref[...] = jnp.dot(jnp.tanh(emb), w_ref[...],
                             preferred_element_type=jnp.float32) * dis

    return pl.pallas_call(
        body,
        grid=(GRID,),
        in_specs=[
            pl.BlockSpec((BN, 128), lambda i: (i, 0)),
            pl.BlockSpec((BN, HC2), lambda i: (i, 0)),
            pl.BlockSpec((BN, 1), lambda i: (i, 0)),
            pl.BlockSpec((1, HC2), lambda i: (0, 0)),
            pl.BlockSpec((HC2, NCLS), lambda i: (0, 0)),
        ],
        out_specs=[
            pl.BlockSpec((BN, HC2), lambda i: (i, 0)),
            pl.BlockSpec((BN, NCLS), lambda i: (i, 0)),
        ],
        out_shape=[
            jax.ShapeDtypeStruct((N, HC2), jnp.float32),
            jax.ShapeDtypeStruct((N_PAD, NCLS), jnp.float32),
        ],
    )(aggp, hs, dis, b, W)


def _tc_final(aggp, hs, dis, b):
    def body(a_ref, hs_ref, dis_ref, b_ref, o_ref):
        a = a_ref[...]
        s = a[:, 0:NCLS] + a[:, NCLS:2 * NCLS] + hs_ref[...]
        o_ref[...] = s * dis_ref[...] + b_ref[...]

    return pl.pallas_call(
        body,
        grid=(GRID,),
        in_specs=[
            pl.BlockSpec((BN, 128), lambda i: (i, 0)),
            pl.BlockSpec((BN, NCLS), lambda i: (i, 0)),
            pl.BlockSpec((BN, 1), lambda i: (i, 0)),
            pl.BlockSpec((1, NCLS), lambda i: (0, 0)),
        ],
        out_specs=pl.BlockSpec((BN, NCLS), lambda i: (i, 0)),
        out_shape=jax.ShapeDtypeStruct((N, NCLS), jnp.float32),
    )(aggp, hs, dis, b)


def _v128(a, C):
    return a.reshape(NCORES, N_PAD * C // 128, 128)


def kernel(x, edge_index, W1, b1, W2, b2, W3, b3):
    eidx = edge_index.reshape(2, NW * N_CHUNKS, CHUNK)

    degp = _make_deg()(eidx)
    h1raw = _tc_mm0(x, W1)
    dis, hs1 = _tc_scale0(h1raw, degp)

    aggp1 = _make_agg(HC1)(hs1, eidx)
    hs2 = _tc_mid(aggp1, hs1, dis, b1.reshape(1, HC1), W2, HC1, HC2)

    aggp2 = _make_agg(HC2)(hs2, eidx)
    emb_p, hs3 = _tc_emb(aggp2, hs2, dis, b2.reshape(1, HC2), W3)

    aggp3 = _make_agg(NCLS)(hs3, eidx)
    logits = _tc_final(aggp3, hs3, dis, b3.reshape(1, NCLS))

    return (logits, emb_p)


# speedup vs baseline: 57.1517x; 1.0020x over previous
import functools

import jax
import jax.numpy as jnp
from jax import lax
from jax.experimental import pallas as pl
from jax.experimental.pallas import tpu as pltpu
from jax.experimental.pallas import tpu_sc as plsc

N = 10000
E = 320000
NF = 128
HC1 = 64
HC2 = 32
NCLS = 16

NCORES = 2
NSUB = 16
NW = NCORES * NSUB
LANES = 16

CHUNK = 125
NBUF = 5
N_CHUNKS = E // (NW * CHUNK)
N_PAD = 10240
ROWS_PER_TILE = N_PAD // NSUB
ZROWS = 128

assert NW * N_CHUNKS * CHUNK == E


def _zero_fill(buf, rows, cols):
    zero = jnp.zeros((LANES,), jnp.float32)
    for r in range(rows):
        for c in range(cols // LANES):
            buf[r, pl.ds(c * LANES, LANES)] = zero


@functools.lru_cache(maxsize=None)
def _make_agg(C):
    NB = NBUF if C >= 64 else 16
    mesh = plsc.VectorSubcoreMesh(core_axis_name="c", subcore_axis_name="s",
                                  num_cores=NCORES, num_subcores=NSUB)

    @functools.partial(
        pl.kernel,
        mesh=mesh,
        out_type=jax.ShapeDtypeStruct((N_PAD, 128), jnp.float32),
        compiler_params=pltpu.CompilerParams(use_tc_tiling_on_sc=False),
        scratch_types=[
            pltpu.VMEM((N_CHUNKS, CHUNK), jnp.int32),
            pltpu.VMEM((N_CHUNKS, CHUNK), jnp.int32),
            [pltpu.VMEM((CHUNK, C), jnp.float32) for _ in range(NB)],
            pltpu.VMEM((ZROWS, C), jnp.float32),
            pltpu.MemorySpace.VMEM_SHARED((N_PAD, C), jnp.float32),
            pltpu.SemaphoreType.DMA,
            [pltpu.SemaphoreType.DMA for _ in range(NB)],
        ],
    )
    def k(hs_hbm, eidx_hbm, out_hbm, sidx, didx, rows, zbuf, acc, isem, bsem):
        cid = lax.axis_index("c")
        sid = lax.axis_index("s")
        wid = cid * NSUB + sid
        r0 = wid * N_CHUNKS
        sload = pltpu.async_copy(eidx_hbm.at[0, pl.ds(r0, N_CHUNKS)], sidx,
                                 isem)
        dload = pltpu.async_copy(eidx_hbm.at[1, pl.ds(r0, N_CHUNKS)], didx,
                                 isem)
        _zero_fill(zbuf, ZROWS, C)
        row0 = sid * ROWS_PER_TILE

        def zloop(i, carry):
            pltpu.sync_copy(zbuf, acc.at[pl.ds(row0 + i * ZROWS, ZROWS)])
            return carry

        lax.fori_loop(0, ROWS_PER_TILE // ZROWS, zloop, 0)
        sload.wait()
        dload.wait()
        plsc.subcore_barrier()

        def gather(i, b):
            pltpu.async_copy(hs_hbm.at[sidx.at[i]], rows[b], bsem[b])

        def scatter(i, b):
            pltpu.async_copy(rows[b], acc.at[didx.at[i]], bsem[b], add=True)

        def wait_b(b):
            pltpu.make_async_copy(hs_hbm.at[sidx.at[0]], rows[b],
                                  bsem[b]).wait()

        def body(g, carry):
            i0 = g * NB
            for b in range(NB):
                pl.when(g > 0)(lambda b=b: wait_b(b))
                gather(i0 + b, b)
            for b in range(NB):
                wait_b(b)
                scatter(i0 + b, b)
            return carry

        lax.fori_loop(0, N_CHUNKS // NB, body, 0)
        for b in range(NB):
            wait_b(b)
        plsc.subcore_barrier()
        pltpu.sync_copy(
            acc.at[pl.ds(row0, ROWS_PER_TILE)],
            out_hbm.at[pl.ds(row0, ROWS_PER_TILE), pl.ds(cid * C, C)],
        )

    return k


@functools.lru_cache(maxsize=None)
def _make_deg():
    C = 16
    NTAIL = (CHUNK // 16) * 16 - (CHUNK - 16)
    mesh = plsc.VectorSubcoreMesh(core_axis_name="c", subcore_axis_name="s",
                                  num_cores=NCORES, num_subcores=NSUB)

    @functools.partial(
        pl.kernel,
        mesh=mesh,
        out_type=jax.ShapeDtypeStruct((N_PAD, 128), jnp.float32),
        compiler_params=pltpu.CompilerParams(use_tc_tiling_on_sc=False,
                                             needs_layout_passes=False),
        scratch_types=[
            pltpu.VMEM((N_CHUNKS, CHUNK), jnp.int32),
            pltpu.VMEM((N_PAD,), jnp.float32),
            pltpu.VMEM((NSUB, ROWS_PER_TILE), jnp.float32),
            pltpu.VMEM((ROWS_PER_TILE, C), jnp.float32),
            pltpu.MemorySpace.VMEM_SHARED((NSUB, N_PAD), jnp.float32),
            pltpu.SemaphoreType.DMA,
        ],
    )
    def k(eidx_hbm, out_hbm, didx, hist, parts, res, shared, isem):
        cid = lax.axis_index("c")
        sid = lax.axis_index("s")
        wid = cid * NSUB + sid
        r0 = wid * N_CHUNKS
        dload = pltpu.async_copy(eidx_hbm.at[1, pl.ds(r0, N_CHUNKS)], didx,
                                 isem)
        zero = jnp.zeros((LANES,), jnp.float32)

        def zloop(i, carry):
            for c in range(16):
                hist[pl.ds(i * 256 + c * 16, 16)] = zero
            return carry

        lax.fori_loop(0, N_PAD // 256, zloop, 0)
        dload.wait()

        ones = jnp.ones((LANES,), jnp.float32)
        lane = lax.iota(jnp.int32, 16)

        def hrow(r, carry):
            for c in range(CHUNK // 16):
                plsc.addupdate_scatter(hist, [didx[r, pl.ds(c * 16, 16)]],
                                       ones)
            plsc.addupdate_scatter(hist, [didx[r, pl.ds(CHUNK - 16, 16)]],
                                   ones, mask=lane >= NTAIL)
            return carry

        lax.fori_loop(0, N_CHUNKS, hrow, 0)
        pltpu.sync_copy(hist, shared.at[sid])
        plsc.subcore_barrier()

        row0 = sid * ROWS_PER_TILE
        pltpu.sync_copy(shared.at[:, pl.ds(row0, ROWS_PER_TILE)], parts)
        col0 = lane * 0

        def cloop(j, carry):
            tot = parts[0, pl.ds(j * 16, 16)]
            for t in range(1, NSUB):
                tot = tot + parts[t, pl.ds(j * 16, 16)]
            plsc.store_scatter(res, [j * 16 + lane, col0], tot)
            return carry

        lax.fori_loop(0, ROWS_PER_TILE // 16, cloop, 0)
        pltpu.sync_copy(
            res,
            out_hbm.at[pl.ds(row0, ROWS_PER_TILE), pl.ds(cid * C, C)],
        )

    return k


BN = 2048
GRID = N_PAD // BN


def _r128(n_rows, C):
    return (n_rows * C // 128, 128)


def _tc_mm0(x, W):
    def body(x_ref, w_ref, o_ref):
        o_ref[...] = jnp.dot(x_ref[...], w_ref[...],
                             preferred_element_type=jnp.float32)

    return pl.pallas_call(
        body,
        grid=(GRID,),
        in_specs=[
            pl.BlockSpec((BN, NF), lambda i: (i, 0)),
            pl.BlockSpec((NF, HC1), lambda i: (0, 0)),
        ],
        out_specs=pl.BlockSpec((BN, HC1), lambda i: (i, 0)),
        out_shape=jax.ShapeDtypeStruct((N_PAD, HC1), jnp.float32),
    )(x, W)


def _tc_scale0(h1raw, degp):
    def body(h_ref, d_ref, dis_ref, hs_ref):
        d = d_ref[...]
        dis = lax.rsqrt(1.0 + d[:, 0:1] + d[:, 16:17])
        dis_ref[...] = dis
        hs_ref[...] = h_ref[...] * dis

    return pl.pallas_call(
        body,
        grid=(GRID,),
        in_specs=[
            pl.BlockSpec((BN, HC1), lambda i: (i, 0)),
            pl.BlockSpec((BN, 128), lambda i: (i, 0)),
        ],
        out_specs=[
            pl.BlockSpec((BN, 1), lambda i: (i, 0)),
            pl.BlockSpec((BN, HC1), lambda i: (i, 0)),
        ],
        out_shape=[
            jax.ShapeDtypeStruct((N_PAD, 1), jnp.float32),
            jax.ShapeDtypeStruct((N_PAD, HC1), jnp.float32),
        ],
    )(h1raw, degp)


def _tc_mid(aggp, hs, dis, b, W, C, C2):
    def body(a_ref, hs_ref, dis_ref, b_ref, w_ref, o_ref):
        a = a_ref[...]
        s = a[:, 0:C] + a[:, C:2 * C] + hs_ref[...]
        dis = dis_ref[...]
        t = jnp.tanh(s * dis + b_ref[...])
        o_ref[...] = jnp.dot(t, w_ref[...],
                             preferred_element_type=jnp.float32) * dis

    return pl.pallas_call(
        body,
        grid=(GRID,),
        in_specs=[
            pl.BlockSpec((BN, 128), lambda i: (i, 0)),
            pl.BlockSpec((BN, C), lambda i: (i, 0)),
            pl.BlockSpec((BN, 1), lambda i: (i, 0)),
            pl.BlockSpec((1, C), lambda i: (0, 0)),
            pl.BlockSpec((C, C2), lambda i: (0, 0)),
        ],
        out_specs=pl.BlockSpec((BN, C2), lambda i: (i, 0)),
        out_shape=jax.ShapeDtypeStruct((N_PAD, C2), jnp.float32),
    )(aggp, hs, dis, b, W)


def _tc_emb(aggp, hs, dis, b, W):
    def body(a_ref, hs_ref, dis_ref, b_ref, w_ref, emb_ref, o_ref):
        a = a_ref[...]
        s = a[:, 0:HC2] + a[:, HC2:2 * HC2] + hs_ref[...]
        dis = dis_ref[...]
        emb = s * dis + b_ref[...]
        emb_ref[...] = emb
        o_ref[...] = jnp.dot(jnp.tanh(emb), w_ref[...],
                             preferred_element_type=jnp.float32) * dis

    return pl.pallas_call(
        body,
        grid=(GRID,),
        in_specs=[
            pl.BlockSpec((BN, 128), lambda i: (i, 0)),
            pl.BlockSpec((BN, HC2), lambda i: (i, 0)),
            pl.BlockSpec((BN, 1), lambda i: (i, 0)),
            pl.BlockSpec((1, HC2), lambda i: (0, 0)),
            pl.BlockSpec((HC2, NCLS), lambda i: (0, 0)),
        ],
        out_specs=[
            pl.BlockSpec((BN, HC2), lambda i: (i, 0)),
            pl.BlockSpec((BN, NCLS), lambda i: (i, 0)),
        ],
        out_shape=[
            jax.ShapeDtypeStruct((N, HC2), jnp.float32),
            jax.ShapeDtypeStruct((N_PAD, NCLS), jnp.float32),
        ],
    )(aggp, hs, dis, b, W)


def _tc_final(aggp, hs, dis, b):
    def body(a_ref, hs_ref, dis_ref, b_ref, o_ref):
        a = a_ref[...]
        s = a[:, 0:NCLS] + a[:, NCLS:2 * NCLS] + hs_ref[...]
        o_ref[...] = s * dis_ref[...] + b_ref[...]

    return pl.pallas_call(
        body,
        grid=(GRID,),
        in_specs=[
            pl.BlockSpec((BN, 128), lambda i: (i, 0)),
            pl.BlockSpec((BN, NCLS), lambda i: (i, 0)),
            pl.BlockSpec((BN, 1), lambda i: (i, 0)),
            pl.BlockSpec((1, NCLS), lambda i: (0, 0)),
        ],
        out_specs=pl.BlockSpec((BN, NCLS), lambda i: (i, 0)),
        out_shape=jax.ShapeDtypeStruct((N, NCLS), jnp.float32),
    )(aggp, hs, dis, b)


def kernel(x, edge_index, W1, b1, W2, b2, W3, b3):
    eidx = edge_index.reshape(2, NW * N_CHUNKS, CHUNK)

    degp = _make_deg()(eidx)
    h1raw = _tc_mm0(x, W1)
    dis, hs1 = _tc_scale0(h1raw, degp)

    aggp1 = _make_agg(HC1)(hs1, eidx)
    hs2 = _tc_mid(aggp1, hs1, dis, b1.reshape(1, HC1), W2, HC1, HC2)

    aggp2 = _make_agg(HC2)(hs2, eidx)
    emb_p, hs3 = _tc_emb(aggp2, hs2, dis, b2.reshape(1, HC2), W3)

    aggp3 = _make_agg(NCLS)(hs3, eidx)
    logits = _tc_final(aggp3, hs3, dis, b3.reshape(1, NCLS))

    return (logits, emb_p)


# speedup vs baseline: 59.5363x; 1.0417x over previous
import functools

import jax
import jax.numpy as jnp
from jax import lax
from jax.experimental import pallas as pl
from jax.experimental.pallas import tpu as pltpu
from jax.experimental.pallas import tpu_sc as plsc

N = 10000
E = 320000
NF = 128
HC1 = 64
HC2 = 32
NCLS = 16

NCORES = 2
NSUB = 16
NW = NCORES * NSUB
LANES = 16

CHUNK = 125
NBUF = 5
N_CHUNKS = E // (NW * CHUNK)
N_PAD = 10240
ROWS_PER_TILE = N_PAD // NSUB
ZROWS = 128

assert NW * N_CHUNKS * CHUNK == E


def _zero_fill(buf, rows, cols):
    zero = jnp.zeros((LANES,), jnp.float32)
    for r in range(rows):
        for c in range(cols // LANES):
            buf[r, pl.ds(c * LANES, LANES)] = zero


@functools.lru_cache(maxsize=None)
def _make_agg(C):
    NB = NBUF if C >= 64 else 16
    mesh = plsc.VectorSubcoreMesh(core_axis_name="c", subcore_axis_name="s",
                                  num_cores=NCORES, num_subcores=NSUB)

    @functools.partial(
        pl.kernel,
        mesh=mesh,
        out_type=jax.ShapeDtypeStruct((N_PAD, 128), jnp.float32),
        compiler_params=pltpu.CompilerParams(use_tc_tiling_on_sc=False),
        scratch_types=[
            pltpu.VMEM((N_CHUNKS, CHUNK), jnp.int32),
            pltpu.VMEM((N_CHUNKS, CHUNK), jnp.int32),
            [pltpu.VMEM((CHUNK, C), jnp.float32) for _ in range(NB)],
            pltpu.VMEM((ZROWS, C), jnp.float32),
            pltpu.MemorySpace.VMEM_SHARED((N_PAD, C), jnp.float32),
            pltpu.SemaphoreType.DMA,
            [pltpu.SemaphoreType.DMA for _ in range(NB)],
        ],
    )
    def k(hs_hbm, eidx_hbm, out_hbm, sidx, didx, rows, zbuf, acc, isem, bsem):
        cid = lax.axis_index("c")
        sid = lax.axis_index("s")
        wid = cid * NSUB + sid
        r0 = wid * N_CHUNKS
        sload = pltpu.async_copy(eidx_hbm.at[0, pl.ds(r0, N_CHUNKS)], sidx,
                                 isem)
        dload = pltpu.async_copy(eidx_hbm.at[1, pl.ds(r0, N_CHUNKS)], didx,
                                 isem)
        _zero_fill(zbuf, ZROWS, C)
        row0 = sid * ROWS_PER_TILE

        def zloop(i, carry):
            pltpu.sync_copy(zbuf, acc.at[pl.ds(row0 + i * ZROWS, ZROWS)])
            return carry

        lax.fori_loop(0, ROWS_PER_TILE // ZROWS, zloop, 0)
        sload.wait()
        dload.wait()
        plsc.subcore_barrier()

        def gather(i, b):
            pltpu.async_copy(hs_hbm.at[sidx.at[i]], rows[b], bsem[b])

        def scatter(i, b):
            pltpu.async_copy(rows[b], acc.at[didx.at[i]], bsem[b], add=True)

        def wait_b(b):
            pltpu.make_async_copy(hs_hbm.at[sidx.at[0]], rows[b],
                                  bsem[b]).wait()

        def body(g, carry):
            i0 = g * NB
            for b in range(NB):
                pl.when(g > 0)(lambda b=b: wait_b(b))
                gather(i0 + b, b)
            for b in range(NB):
                wait_b(b)
                scatter(i0 + b, b)
            return carry

        lax.fori_loop(0, N_CHUNKS // NB, body, 0)
        for b in range(NB):
            wait_b(b)
        plsc.subcore_barrier()
        pltpu.sync_copy(
            acc.at[pl.ds(row0, ROWS_PER_TILE)],
            out_hbm.at[pl.ds(row0, ROWS_PER_TILE), pl.ds(cid * C, C)],
        )

    return k


@functools.lru_cache(maxsize=None)
def _make_deg():
    C = 16
    NTAIL = (CHUNK // 16) * 16 - (CHUNK - 16)
    mesh = plsc.VectorSubcoreMesh(core_axis_name="c", subcore_axis_name="s",
                                  num_cores=NCORES, num_subcores=NSUB)

    @functools.partial(
        pl.kernel,
        mesh=mesh,
        out_type=jax.ShapeDtypeStruct((N_PAD, 128), jnp.float32),
        compiler_params=pltpu.CompilerParams(use_tc_tiling_on_sc=False,
                                             needs_layout_passes=False),
        scratch_types=[
            pltpu.VMEM((N_CHUNKS, CHUNK), jnp.int32),
            pltpu.VMEM((N_PAD,), jnp.float32),
            pltpu.VMEM((NSUB, ROWS_PER_TILE), jnp.float32),
            pltpu.VMEM((ROWS_PER_TILE, C), jnp.float32),
            pltpu.MemorySpace.VMEM_SHARED((NSUB, N_PAD), jnp.float32),
            pltpu.SemaphoreType.DMA,
        ],
    )
    def k(eidx_hbm, out_hbm, didx, hist, parts, res, shared, isem):
        cid = lax.axis_index("c")
        sid = lax.axis_index("s")
        wid = cid * NSUB + sid
        r0 = wid * N_CHUNKS
        dload = pltpu.async_copy(eidx_hbm.at[1, pl.ds(r0, N_CHUNKS)], didx,
                                 isem)
        zero = jnp.zeros((LANES,), jnp.float32)

        def zloop(i, carry):
            for c in range(16):
                hist[pl.ds(i * 256 + c * 16, 16)] = zero
            return carry

        lax.fori_loop(0, N_PAD // 256, zloop, 0)
        dload.wait()

        ones = jnp.ones((LANES,), jnp.float32)
        lane = lax.iota(jnp.int32, 16)

        def hrow(r, carry):
            for c in range(CHUNK // 16):
                plsc.addupdate_scatter(hist, [didx[r, pl.ds(c * 16, 16)]],
                                       ones)
            plsc.addupdate_scatter(hist, [didx[r, pl.ds(CHUNK - 16, 16)]],
                                   ones, mask=lane >= NTAIL)
            return carry

        lax.fori_loop(0, N_CHUNKS, hrow, 0)
        pltpu.sync_copy(hist, shared.at[sid])
        plsc.subcore_barrier()

        row0 = sid * ROWS_PER_TILE
        pltpu.sync_copy(shared.at[:, pl.ds(row0, ROWS_PER_TILE)], parts)
        col0 = lane * 0

        def cloop(j, carry):
            tot = parts[0, pl.ds(j * 16, 16)]
            for t in range(1, NSUB):
                tot = tot + parts[t, pl.ds(j * 16, 16)]
            plsc.store_scatter(res, [j * 16 + lane, col0], tot)
            return carry

        lax.fori_loop(0, ROWS_PER_TILE // 16, cloop, 0)
        pltpu.sync_copy(
            res,
            out_hbm.at[pl.ds(row0, ROWS_PER_TILE), pl.ds(cid * C, C)],
        )

    return k


BN = 5120
GRID = N_PAD // BN


def _r128(n_rows, C):
    return (n_rows * C // 128, 128)


def _tc_mm0(x, W):
    def body(x_ref, w_ref, o_ref):
        o_ref[...] = jnp.dot(x_ref[...], w_ref[...],
                             preferred_element_type=jnp.float32)

    return pl.pallas_call(
        body,
        grid=(GRID,),
        in_specs=[
            pl.BlockSpec((BN, NF), lambda i: (i, 0)),
            pl.BlockSpec((NF, HC1), lambda i: (0, 0)),
        ],
        out_specs=pl.BlockSpec((BN, HC1), lambda i: (i, 0)),
        out_shape=jax.ShapeDtypeStruct((N_PAD, HC1), jnp.float32),
    )(x, W)


def _tc_scale0(h1raw, degp):
    def body(h_ref, d_ref, dis_ref, hs_ref):
        d = d_ref[...]
        dis = lax.rsqrt(1.0 + d[:, 0:1] + d[:, 16:17])
        dis_ref[...] = dis
        hs_ref[...] = h_ref[...] * dis

    return pl.pallas_call(
        body,
        grid=(GRID,),
        in_specs=[
            pl.BlockSpec((BN, HC1), lambda i: (i, 0)),
            pl.BlockSpec((BN, 128), lambda i: (i, 0)),
        ],
        out_specs=[
            pl.BlockSpec((BN, 1), lambda i: (i, 0)),
            pl.BlockSpec((BN, HC1), lambda i: (i, 0)),
        ],
        out_shape=[
            jax.ShapeDtypeStruct((N_PAD, 1), jnp.float32),
            jax.ShapeDtypeStruct((N_PAD, HC1), jnp.float32),
        ],
    )(h1raw, degp)


def _tc_mid(aggp, hs, dis, b, W, C, C2):
    def body(a_ref, hs_ref, dis_ref, b_ref, w_ref, o_ref):
        a = a_ref[...]
        s = a[:, 0:C] + a[:, C:2 * C] + hs_ref[...]
        dis = dis_ref[...]
        t = jnp.tanh(s * dis + b_ref[...])
        o_ref[...] = jnp.dot(t, w_ref[...],
                             preferred_element_type=jnp.float32) * dis

    return pl.pallas_call(
        body,
        grid=(GRID,),
        in_specs=[
            pl.BlockSpec((BN, 128), lambda i: (i, 0)),
            pl.BlockSpec((BN, C), lambda i: (i, 0)),
            pl.BlockSpec((BN, 1), lambda i: (i, 0)),
            pl.BlockSpec((1, C), lambda i: (0, 0)),
            pl.BlockSpec((C, C2), lambda i: (0, 0)),
        ],
        out_specs=pl.BlockSpec((BN, C2), lambda i: (i, 0)),
        out_shape=jax.ShapeDtypeStruct((N_PAD, C2), jnp.float32),
    )(aggp, hs, dis, b, W)


def _tc_emb(aggp, hs, dis, b, W):
    def body(a_ref, hs_ref, dis_ref, b_ref, w_ref, emb_ref, o_ref):
        a = a_ref[...]
        s = a[:, 0:HC2] + a[:, HC2:2 * HC2] + hs_ref[...]
        dis = dis_ref[...]
        emb = s * dis + b_ref[...]
        emb_ref[...] = emb
        o_ref[...] = jnp.dot(jnp.tanh(emb), w_ref[...],
                             preferred_element_type=jnp.float32) * dis

    return pl.pallas_call(
        body,
        grid=(GRID,),
        in_specs=[
            pl.BlockSpec((BN, 128), lambda i: (i, 0)),
            pl.BlockSpec((BN, HC2), lambda i: (i, 0)),
            pl.BlockSpec((BN, 1), lambda i: (i, 0)),
            pl.BlockSpec((1, HC2), lambda i: (0, 0)),
            pl.BlockSpec((HC2, NCLS), lambda i: (0, 0)),
        ],
        out_specs=[
            pl.BlockSpec((BN, HC2), lambda i: (i, 0)),
            pl.BlockSpec((BN, NCLS), lambda i: (i, 0)),
        ],
        out_shape=[
            jax.ShapeDtypeStruct((N, HC2), jnp.float32),
            jax.ShapeDtypeStruct((N_PAD, NCLS), jnp.float32),
        ],
    )(aggp, hs, dis, b, W)


def _tc_final(aggp, hs, dis, b):
    def body(a_ref, hs_ref, dis_ref, b_ref, o_ref):
        a = a_ref[...]
        s = a[:, 0:NCLS] + a[:, NCLS:2 * NCLS] + hs_ref[...]
        o_ref[...] = s * dis_ref[...] + b_ref[...]

    return pl.pallas_call(
        body,
        grid=(GRID,),
        in_specs=[
            pl.BlockSpec((BN, 128), lambda i: (i, 0)),
            pl.BlockSpec((BN, NCLS), lambda i: (i, 0)),
            pl.BlockSpec((BN, 1), lambda i: (i, 0)),
            pl.BlockSpec((1, NCLS), lambda i: (0, 0)),
        ],
        out_specs=pl.BlockSpec((BN, NCLS), lambda i: (i, 0)),
        out_shape=jax.ShapeDtypeStruct((N, NCLS), jnp.float32),
    )(aggp, hs, dis, b)


def kernel(x, edge_index, W1, b1, W2, b2, W3, b3):
    eidx = edge_index.reshape(2, NW * N_CHUNKS, CHUNK)

    degp = _make_deg()(eidx)
    h1raw = _tc_mm0(x, W1)
    dis, hs1 = _tc_scale0(h1raw, degp)

    aggp1 = _make_agg(HC1)(hs1, eidx)
    hs2 = _tc_mid(aggp1, hs1, dis, b1.reshape(1, HC1), W2, HC1, HC2)

    aggp2 = _make_agg(HC2)(hs2, eidx)
    emb_p, hs3 = _tc_emb(aggp2, hs2, dis, b2.reshape(1, HC2), W3)

    aggp3 = _make_agg(NCLS)(hs3, eidx)
    logits = _tc_final(aggp3, hs3, dis, b3.reshape(1, NCLS))

    return (logits, emb_p)
